# Initial kernel scaffold; baseline (speedup 1.0000x reference)
import jax
import jax.numpy as jnp
from jax.experimental import pallas as pl


def kernel(query_node, key_edge, value_edge, edge_index, Wq, Wk, Wv, Wo, bo, gamma, beta):
    raise NotImplementedError("write your pallas kernel here")


# speedup vs baseline: 6.7599x; 6.7599x over previous
import jax
import jax.numpy as jnp
from jax import lax
from jax.experimental import pallas as pl
from jax.experimental.pallas import tpu as pltpu
from jax.experimental.pallas import tpu_sc as plsc

N = 10000
E = 320000
D = 128
NC = 2
NS = 16
NW = NC * NS
EP = E // NW
C = 128
FC = EP // C
TAIL = EP - FC * C
PAD_N = 10240
RPS = PAD_N // NS
RPW = PAD_N // NW
SCALE = D ** -0.5

_F32 = jnp.float32
_I32 = jnp.int32


def _mesh():
    return plsc.VectorSubcoreMesh(
        core_axis_name="c", subcore_axis_name="s",
        num_cores=NC, num_subcores=NS)


_DNUMS = lax.GatherDimensionNumbers(
    offset_dims=(), collapsed_slice_dims=(0,), start_index_map=(0,))


def _perm(v, idx):
    return lax.gather(v, idx[:, None], _DNUMS, (1,),
                      mode=lax.GatherScatterMode.PROMISE_IN_BOUNDS)


def _hsum(v, lane):
    for sh in (8, 4, 2, 1):
        v = v + _perm(v, lane ^ sh)
    return v


def _hmax(v, lane):
    for sh in (8, 4, 2, 1):
        v = jnp.maximum(v, _perm(v, lane ^ sh))
    return v


def _dot(a, b, dims):
    return lax.dot_general(
        a, b, (dims, ((), ())),
        precision=lax.Precision.HIGHEST,
        preferred_element_type=_F32)


def _p_body(x_ref, wq_ref, wk_ref, o_ref):
    t = _dot(x_ref[...], wq_ref[...], ((1,), (0,)))
    o_ref[...] = _dot(t, wk_ref[...], ((1,), (1,))) * SCALE


def _compute_p(query, Wq, Wk):
    return pl.pallas_call(
        _p_body,
        grid=(25,),
        in_specs=[pl.BlockSpec((400, D), lambda i: (i, 0)),
                  pl.BlockSpec((D, D), lambda i: (0, 0)),
                  pl.BlockSpec((D, D), lambda i: (0, 0))],
        out_specs=pl.BlockSpec((400, D), lambda i: (i, 0)),
        out_shape=jax.ShapeDtypeStruct((N, D), _F32),
    )(query, Wq, Wk)


def _sc_scores_body(p_hbm, key_hbm, tgt_hbm, s_hbm, mx_hbm,
                    idx_v, pg_v, key_v, s_v,
                    idx_t, pg_t, key_t, s_t, mx_v):
    cid = lax.axis_index("c")
    sid = lax.axis_index("s")
    w = cid * NS + sid
    ebase = w * EP
    lane = lax.iota(_I32, 16)

    def run_chunk(base, idxr, pgr, keyr, sr, n_edges, mv):
        pltpu.sync_copy(tgt_hbm.at[pl.ds(base, n_edges)], idxr)
        pltpu.sync_copy(key_hbm.at[pl.ds(base, n_edges)], keyr)
        pltpu.sync_copy(p_hbm.at[idxr], pgr)

        def grp(g, mv):
            svec = jnp.zeros((16,), _F32)
            for u in range(16):
                i = g * 16 + u
                acc = pgr[i, pl.ds(0, 16)] * keyr[i, pl.ds(0, 16)]
                for j in range(1, 8):
                    acc = acc + pgr[i, pl.ds(16 * j, 16)] * keyr[i, pl.ds(16 * j, 16)]
                svec = jnp.where(lane == u, _hsum(acc, lane), svec)
            sr[pl.ds(g * 16, 16)] = svec
            return jnp.maximum(mv, svec)

        mv = lax.fori_loop(0, n_edges // 16, grp, mv)
        pltpu.sync_copy(sr, s_hbm.at[pl.ds(base, n_edges)])
        return mv

    def chunk(ci, mv):
        return run_chunk(ebase + ci * C, idx_v, pg_v, key_v, s_v, C, mv)

    mv = lax.fori_loop(0, FC, chunk, jnp.full((16,), -1e30, _F32))
    mv = run_chunk(ebase + FC * C, idx_t, pg_t, key_t, s_t, TAIL, mv)
    mx_v[...] = mv
    pltpu.sync_copy(mx_v, mx_hbm.at[pl.ds(w * 16, 16)])


def _sc_scores(p, key_edge, tgt):
    f = pl.kernel(
        _sc_scores_body,
        out_type=(jax.ShapeDtypeStruct((E,), _F32),
                  jax.ShapeDtypeStruct((NW * 16,), _F32)),
        mesh=_mesh(),
        scratch_types=[
            pltpu.VMEM((C,), _I32),
            pltpu.VMEM((C, D), _F32),
            pltpu.VMEM((C, D), _F32),
            pltpu.VMEM((C,), _F32),
            pltpu.VMEM((TAIL,), _I32),
            pltpu.VMEM((TAIL, D), _F32),
            pltpu.VMEM((TAIL, D), _F32),
            pltpu.VMEM((TAIL,), _F32),
            pltpu.VMEM((16,), _F32),
        ])
    return f(p, key_edge, tgt)


def _sc_agg_body(s_hbm, mx_hbm, tgt_hbm, val_hbm,
                 e_hbm, agg_hbm, sums_hbm,
                 idx_v, s_v, e_v, val_v,
                 idx_t, s_t, e_t, val_t,
                 mxl_v, zrow_v, zsum_v,
                 agg_sh, sums_sh):
    cid = lax.axis_index("c")
    sid = lax.axis_index("s")
    w = cid * NS + sid
    ebase = w * EP
    lane = lax.iota(_I32, 16)

    def zr(r, _):
        for j in range(8):
            zrow_v[r, pl.ds(16 * j, 16)] = jnp.zeros((16,), _F32)
        return 0
    lax.fori_loop(0, 80, zr, 0)

    def zs(i, _):
        zsum_v[pl.ds(16 * i, 16)] = jnp.zeros((16,), _F32)
        return 0
    lax.fori_loop(0, RPS // 16, zs, 0)

    for k2 in range(RPS // 80):
        pltpu.sync_copy(zrow_v, agg_sh.at[pl.ds(sid * RPS + k2 * 80, 80)])
    pltpu.sync_copy(zsum_v, sums_sh.at[pl.ds(sid * RPS, RPS)])
    plsc.subcore_barrier()

    pltpu.sync_copy(mx_hbm, mxl_v)

    def mred(i, mv):
        return jnp.maximum(mv, mxl_v[pl.ds(16 * i, 16)])
    mv = lax.fori_loop(0, NW, mred, jnp.full((16,), -1e30, _F32))
    m_shift = _hmax(mv, lane)

    def run_chunk(base, idxr, sr, er, vr, n_edges):
        pltpu.sync_copy(tgt_hbm.at[pl.ds(base, n_edges)], idxr)
        pltpu.sync_copy(s_hbm.at[pl.ds(base, n_edges)], sr)
        pltpu.sync_copy(val_hbm.at[pl.ds(base, n_edges)], vr)

        def grp(g, _):
            sl = pl.ds(g * 16, 16)
            ev16 = jnp.exp(sr[sl] - m_shift)
            er[sl] = ev16
            for u in range(16):
                i = g * 16 + u
                wb = _perm(ev16, jnp.full((16,), u, _I32))
                for j in range(8):
                    slj = pl.ds(16 * j, 16)
                    vr[i, slj] = vr[i, slj] * wb
            return 0
        lax.fori_loop(0, n_edges // 16, grp, 0)

        pltpu.sync_copy(er, e_hbm.at[pl.ds(base, n_edges)])
        pltpu.sync_copy(vr, agg_sh.at[idxr], add=True)
        pltpu.sync_copy(er, sums_sh.at[idxr], add=True)

    def chunk(ci, _):
        run_chunk(ebase + ci * C, idx_v, s_v, e_v, val_v, C)
        return 0
    lax.fori_loop(0, FC, chunk, 0)
    run_chunk(ebase + FC * C, idx_t, s_t, e_t, val_t, TAIL)

    plsc.subcore_barrier()
    pltpu.sync_copy(agg_sh.at[pl.ds(sid * RPS, RPS)],
                    agg_hbm.at[cid, pl.ds(sid * RPS, RPS)])
    pltpu.sync_copy(sums_sh.at[pl.ds(sid * RPS, RPS)],
                    sums_hbm.at[cid, pl.ds(sid * RPS, RPS)])


def _sc_agg(s, mx, tgt, value_edge):
    f = pl.kernel(
        _sc_agg_body,
        out_type=(jax.ShapeDtypeStruct((E,), _F32),
                  jax.ShapeDtypeStruct((NC, PAD_N, D), _F32),
                  jax.ShapeDtypeStruct((NC, PAD_N), _F32)),
        mesh=_mesh(),
        scratch_types=[
            pltpu.VMEM((C,), _I32),
            pltpu.VMEM((C,), _F32),
            pltpu.VMEM((C,), _F32),
            pltpu.VMEM((C, D), _F32),
            pltpu.VMEM((TAIL,), _I32),
            pltpu.VMEM((TAIL,), _F32),
            pltpu.VMEM((TAIL,), _F32),
            pltpu.VMEM((TAIL, D), _F32),
            pltpu.VMEM((NW * 16,), _F32),
            pltpu.VMEM((80, D), _F32),
            pltpu.VMEM((RPS,), _F32),
            pltpu.VMEM_SHARED((PAD_N, D), _F32),
            pltpu.VMEM_SHARED((PAD_N,), _F32),
        ])
    return f(s, mx, tgt, value_edge)


def _rn_body(s_ref, o_ref):
    o_ref[...] = 1.0 / (s_ref[0] + s_ref[1] + 1e-12)


def _compute_rn(sums2):
    return pl.pallas_call(
        _rn_body,
        out_shape=jax.ShapeDtypeStruct((PAD_N // D, D), _F32),
    )(sums2.reshape(NC, PAD_N // D, D))


def _sc_w_body(e_hbm, tgt_hbm, rn_hbm, w_hbm,
               idx_v, e_v, rg_v, w_v, idx_t, e_t, rg_t, w_t):
    cid = lax.axis_index("c")
    sid = lax.axis_index("s")
    w = cid * NS + sid
    ebase = w * EP

    def run_chunk(base, idxr, er, rgr, wr, n_edges):
        pltpu.sync_copy(tgt_hbm.at[pl.ds(base, n_edges)], idxr)
        pltpu.sync_copy(e_hbm.at[pl.ds(base, n_edges)], er)
        pltpu.sync_copy(rn_hbm.at[idxr], rgr)

        def grp(g, _):
            sl = pl.ds(g * 16, 16)
            wr[sl] = er[sl] * rgr[sl]
            return 0
        lax.fori_loop(0, n_edges // 16, grp, 0)
        pltpu.sync_copy(wr, w_hbm.at[pl.ds(base, n_edges)])

    def chunk(ci, _):
        run_chunk(ebase + ci * C, idx_v, e_v, rg_v, w_v, C)
        return 0
    lax.fori_loop(0, FC, chunk, 0)
    run_chunk(ebase + FC * C, idx_t, e_t, rg_t, w_t, TAIL)


def _sc_weights(e, tgt, rn_flat):
    f = pl.kernel(
        _sc_w_body,
        out_type=jax.ShapeDtypeStruct((E,), _F32),
        mesh=_mesh(),
        scratch_types=[
            pltpu.VMEM((C,), _I32),
            pltpu.VMEM((C,), _F32),
            pltpu.VMEM((C,), _F32),
            pltpu.VMEM((C,), _F32),
            pltpu.VMEM((TAIL,), _I32),
            pltpu.VMEM((TAIL,), _F32),
            pltpu.VMEM((TAIL,), _F32),
            pltpu.VMEM((TAIL,), _F32),
        ])
    return f(e, tgt, rn_flat)


def _out_body(a_ref, rn_ref, wv_ref, wo_ref, bo_ref, g_ref, b_ref, o_ref):
    a_n = (a_ref[0] + a_ref[1]) * rn_ref[...]
    t = _dot(a_n, wv_ref[...], ((1,), (0,)))
    proj = _dot(t, wo_ref[...], ((1,), (0,))) + bo_ref[...]
    mu = jnp.mean(proj, axis=1, keepdims=True)
    cen = proj - mu
    var = jnp.mean(cen * cen, axis=1, keepdims=True)
    o_ref[...] = cen * lax.rsqrt(var + 1e-5) * g_ref[...] + b_ref[...]


def _finalize(agg2, rn_col, Wv, Wo, bo, gamma, beta):
    return pl.pallas_call(
        _out_body,
        grid=(PAD_N // 256,),
        in_specs=[pl.BlockSpec((NC, 256, D), lambda i: (0, i, 0)),
                  pl.BlockSpec((256, 1), lambda i: (i, 0)),
                  pl.BlockSpec((D, D), lambda i: (0, 0)),
                  pl.BlockSpec((D, D), lambda i: (0, 0)),
                  pl.BlockSpec((1, D), lambda i: (0, 0)),
                  pl.BlockSpec((1, D), lambda i: (0, 0)),
                  pl.BlockSpec((1, D), lambda i: (0, 0))],
        out_specs=pl.BlockSpec((256, D), lambda i: (i, 0)),
        out_shape=jax.ShapeDtypeStruct((PAD_N, D), _F32),
    )(agg2, rn_col, Wv, Wo, bo.reshape(1, D), gamma.reshape(1, D), beta.reshape(1, D))


def kernel(query_node, key_edge, value_edge, edge_index, Wq, Wk, Wv, Wo, bo, gamma, beta):
    tgt = edge_index[1]
    p = _compute_p(query_node, Wq, Wk)
    s, mx = _sc_scores(p, key_edge, tgt)
    e, agg2, sums2 = _sc_agg(s, mx, tgt, value_edge)
    rn = _compute_rn(sums2)
    wts = _sc_weights(e, tgt, rn.reshape(PAD_N))
    out = _finalize(agg2, rn.reshape(PAD_N, 1), Wv, Wo, bo, gamma, beta)
    return out[:N], wts


# speedup vs baseline: 7.6551x; 1.1324x over previous
import jax
import jax.numpy as jnp
from jax import lax
from jax.experimental import pallas as pl
from jax.experimental.pallas import tpu as pltpu
from jax.experimental.pallas import tpu_sc as plsc

N = 10000
E = 320000
D = 128
NC = 2
NS = 16
NW = NC * NS
EP = E // NW
C = 128
FC = EP // C
TAIL = EP - FC * C
PAD_N = 10240
RPS = PAD_N // NS
RPW = PAD_N // NW
SCALE = D ** -0.5

_F32 = jnp.float32
_I32 = jnp.int32


def _mesh():
    return plsc.VectorSubcoreMesh(
        core_axis_name="c", subcore_axis_name="s",
        num_cores=NC, num_subcores=NS)


_DNUMS = lax.GatherDimensionNumbers(
    offset_dims=(), collapsed_slice_dims=(0,), start_index_map=(0,))


def _perm(v, idx):
    return lax.gather(v, idx[:, None], _DNUMS, (1,),
                      mode=lax.GatherScatterMode.PROMISE_IN_BOUNDS)


def _hsum(v, lane):
    for sh in (8, 4, 2, 1):
        v = v + _perm(v, lane ^ sh)
    return v


def _hmax(v, lane):
    for sh in (8, 4, 2, 1):
        v = jnp.maximum(v, _perm(v, lane ^ sh))
    return v


def _dot(a, b, dims):
    return lax.dot_general(
        a, b, (dims, ((), ())),
        precision=lax.Precision.HIGHEST,
        preferred_element_type=_F32)


def _p_body(x_ref, wq_ref, wk_ref, o_ref):
    t = _dot(x_ref[...], wq_ref[...], ((1,), (0,)))
    o_ref[...] = _dot(t, wk_ref[...], ((1,), (1,))) * SCALE


def _compute_p(query, Wq, Wk):
    return pl.pallas_call(
        _p_body,
        grid=(25,),
        in_specs=[pl.BlockSpec((400, D), lambda i: (i, 0)),
                  pl.BlockSpec((D, D), lambda i: (0, 0)),
                  pl.BlockSpec((D, D), lambda i: (0, 0))],
        out_specs=pl.BlockSpec((400, D), lambda i: (i, 0)),
        out_shape=jax.ShapeDtypeStruct((N, D), _F32),
    )(query, Wq, Wk)


def _sc_edge_body(p_hbm, key_hbm, tgt_hbm, e_hbm,
                  idx_v, pg_v, key_v, e_v,
                  idx_t, pg_t, key_t, e_t,
                  sem_i, sem_k, sem_g):
    cid = lax.axis_index("c")
    sid = lax.axis_index("s")
    w = cid * NS + sid
    ebase = w * EP
    lane = lax.iota(_I32, 16)

    def run_chunk(base, idxr, pgr, keyr, er, n_edges):
        ci = pltpu.async_copy(tgt_hbm.at[pl.ds(base, n_edges)], idxr, sem_i)
        ck = pltpu.async_copy(key_hbm.at[pl.ds(base, n_edges)], keyr, sem_k)
        ci.wait()
        cg = pltpu.async_copy(p_hbm.at[idxr], pgr, sem_g)
        ck.wait()
        cg.wait()

        def grp(g, _):
            svec = jnp.zeros((16,), _F32)
            for u in range(16):
                i = g * 16 + u
                pr = [pgr[i, pl.ds(16 * j, 16)] * keyr[i, pl.ds(16 * j, 16)]
                      for j in range(8)]
                acc = ((pr[0] + pr[1]) + (pr[2] + pr[3])) + \
                      ((pr[4] + pr[5]) + (pr[6] + pr[7]))
                svec = jnp.where(lane == u, _hsum(acc, lane), svec)
            er[pl.ds(g * 16, 16)] = jnp.exp(jnp.minimum(svec, 60.0))
            return 0
        lax.fori_loop(0, n_edges // 16, grp, 0)
        pltpu.sync_copy(er, e_hbm.at[pl.ds(base, n_edges)])

    def chunk(ci_, _):
        run_chunk(ebase + ci_ * C, idx_v, pg_v, key_v, e_v, C)
        return 0
    lax.fori_loop(0, FC, chunk, 0)
    run_chunk(ebase + FC * C, idx_t, pg_t, key_t, e_t, TAIL)


def _sc_edge(p, key_edge, tgt):
    f = pl.kernel(
        _sc_edge_body,
        out_type=jax.ShapeDtypeStruct((E,), _F32),
        mesh=_mesh(),
        scratch_types=[
            pltpu.VMEM((C,), _I32),
            pltpu.VMEM((C, D), _F32),
            pltpu.VMEM((C, D), _F32),
            pltpu.VMEM((C,), _F32),
            pltpu.VMEM((TAIL,), _I32),
            pltpu.VMEM((TAIL, D), _F32),
            pltpu.VMEM((TAIL, D), _F32),
            pltpu.VMEM((TAIL,), _F32),
            pltpu.SemaphoreType.DMA,
            pltpu.SemaphoreType.DMA,
            pltpu.SemaphoreType.DMA,
        ])
    return f(p, key_edge, tgt)


def _sc_scatter_body(e_hbm, tgt_hbm, val_hbm,
                     agg_hbm, sums_hbm,
                     idx_v, e_v, val_v, idx_t, e_t, val_t,
                     zrow_v, zsum_v,
                     sem_i, sem_e, sem_v, sem_s1, sem_s2,
                     agg_sh, sums_sh):
    cid = lax.axis_index("c")
    sid = lax.axis_index("s")
    w = cid * NS + sid
    ebase = w * EP
    lane = lax.iota(_I32, 16)

    def zr(r, _):
        for j in range(8):
            zrow_v[r, pl.ds(16 * j, 16)] = jnp.zeros((16,), _F32)
        return 0
    lax.fori_loop(0, 40, zr, 0)

    def zs(i, _):
        zsum_v[pl.ds(16 * i, 16)] = jnp.zeros((16,), _F32)
        return 0
    lax.fori_loop(0, RPS // 16, zs, 0)

    for k2 in range(RPS // 40):
        pltpu.sync_copy(zrow_v, agg_sh.at[pl.ds(sid * RPS + k2 * 40, 40)])
    pltpu.sync_copy(zsum_v, sums_sh.at[pl.ds(sid * RPS, RPS)])
    plsc.subcore_barrier()

    def run_chunk(base, idxr, er, vr, n_edges):
        c1 = pltpu.async_copy(tgt_hbm.at[pl.ds(base, n_edges)], idxr, sem_i)
        c2 = pltpu.async_copy(e_hbm.at[pl.ds(base, n_edges)], er, sem_e)
        c3 = pltpu.async_copy(val_hbm.at[pl.ds(base, n_edges)], vr, sem_v)
        c1.wait()
        c2.wait()
        c3.wait()

        def grp(g, _):
            ev16 = er[pl.ds(g * 16, 16)]
            for u in range(16):
                i = g * 16 + u
                wb = _perm(ev16, jnp.full((16,), u, _I32))
                for j in range(8):
                    slj = pl.ds(16 * j, 16)
                    vr[i, slj] = vr[i, slj] * wb
            return 0
        lax.fori_loop(0, n_edges // 16, grp, 0)

        cs1 = pltpu.async_copy(vr, agg_sh.at[idxr], sem_s1, add=True)
        cs2 = pltpu.async_copy(er, sums_sh.at[idxr], sem_s2, add=True)
        cs1.wait()
        cs2.wait()

    def chunk(ci, _):
        run_chunk(ebase + ci * C, idx_v, e_v, val_v, C)
        return 0
    lax.fori_loop(0, FC, chunk, 0)
    run_chunk(ebase + FC * C, idx_t, e_t, val_t, TAIL)

    plsc.subcore_barrier()
    pltpu.sync_copy(agg_sh.at[pl.ds(sid * RPS, RPS)],
                    agg_hbm.at[cid, pl.ds(sid * RPS, RPS)])
    pltpu.sync_copy(sums_sh.at[pl.ds(sid * RPS, RPS)],
                    sums_hbm.at[cid, pl.ds(sid * RPS, RPS)])


def _sc_scatter(e, tgt, value_edge):
    f = pl.kernel(
        _sc_scatter_body,
        out_type=(jax.ShapeDtypeStruct((NC, PAD_N, D), _F32),
                  jax.ShapeDtypeStruct((NC, PAD_N), _F32)),
        mesh=_mesh(),
        scratch_types=[
            pltpu.VMEM((C,), _I32),
            pltpu.VMEM((C,), _F32),
            pltpu.VMEM((C, D), _F32),
            pltpu.VMEM((TAIL,), _I32),
            pltpu.VMEM((TAIL,), _F32),
            pltpu.VMEM((TAIL, D), _F32),
            pltpu.VMEM((40, D), _F32),
            pltpu.VMEM((RPS,), _F32),
            pltpu.SemaphoreType.DMA,
            pltpu.SemaphoreType.DMA,
            pltpu.SemaphoreType.DMA,
            pltpu.SemaphoreType.DMA,
            pltpu.SemaphoreType.DMA,
            pltpu.VMEM_SHARED((PAD_N, D), _F32),
            pltpu.VMEM_SHARED((PAD_N,), _F32),
        ])
    return f(e, tgt, value_edge)


def _rn_body(s_ref, o_ref):
    o_ref[...] = 1.0 / (s_ref[0] + s_ref[1] + 1e-12)


def _compute_rn(sums2):
    return pl.pallas_call(
        _rn_body,
        out_shape=jax.ShapeDtypeStruct((PAD_N // D, D), _F32),
    )(sums2.reshape(NC, PAD_N // D, D))


def _sc_w_body(e_hbm, tgt_hbm, rn_hbm, w_hbm,
               idx_v, e_v, rg_v, w_v, idx_t, e_t, rg_t, w_t):
    cid = lax.axis_index("c")
    sid = lax.axis_index("s")
    w = cid * NS + sid
    ebase = w * EP

    def run_chunk(base, idxr, er, rgr, wr, n_edges):
        pltpu.sync_copy(tgt_hbm.at[pl.ds(base, n_edges)], idxr)
        pltpu.sync_copy(e_hbm.at[pl.ds(base, n_edges)], er)
        pltpu.sync_copy(rn_hbm.at[idxr], rgr)

        def grp(g, _):
            sl = pl.ds(g * 16, 16)
            wr[sl] = er[sl] * rgr[sl]
            return 0
        lax.fori_loop(0, n_edges // 16, grp, 0)
        pltpu.sync_copy(wr, w_hbm.at[pl.ds(base, n_edges)])

    def chunk(ci, _):
        run_chunk(ebase + ci * C, idx_v, e_v, rg_v, w_v, C)
        return 0
    lax.fori_loop(0, FC, chunk, 0)
    run_chunk(ebase + FC * C, idx_t, e_t, rg_t, w_t, TAIL)


def _sc_weights(e, tgt, rn_flat):
    f = pl.kernel(
        _sc_w_body,
        out_type=jax.ShapeDtypeStruct((E,), _F32),
        mesh=_mesh(),
        scratch_types=[
            pltpu.VMEM((C,), _I32),
            pltpu.VMEM((C,), _F32),
            pltpu.VMEM((C,), _F32),
            pltpu.VMEM((C,), _F32),
            pltpu.VMEM((TAIL,), _I32),
            pltpu.VMEM((TAIL,), _F32),
            pltpu.VMEM((TAIL,), _F32),
            pltpu.VMEM((TAIL,), _F32),
        ])
    return f(e, tgt, rn_flat)


def _out_body(a_ref, rn_ref, wv_ref, wo_ref, bo_ref, g_ref, b_ref, o_ref):
    a_n = (a_ref[0] + a_ref[1]) * rn_ref[...]
    t = _dot(a_n, wv_ref[...], ((1,), (0,)))
    proj = _dot(t, wo_ref[...], ((1,), (0,))) + bo_ref[...]
    mu = jnp.mean(proj, axis=1, keepdims=True)
    cen = proj - mu
    var = jnp.mean(cen * cen, axis=1, keepdims=True)
    o_ref[...] = cen * lax.rsqrt(var + 1e-5) * g_ref[...] + b_ref[...]


def _finalize(agg2, rn_col, Wv, Wo, bo, gamma, beta):
    return pl.pallas_call(
        _out_body,
        grid=(PAD_N // 256,),
        in_specs=[pl.BlockSpec((NC, 256, D), lambda i: (0, i, 0)),
                  pl.BlockSpec((256, 1), lambda i: (i, 0)),
                  pl.BlockSpec((D, D), lambda i: (0, 0)),
                  pl.BlockSpec((D, D), lambda i: (0, 0)),
                  pl.BlockSpec((1, D), lambda i: (0, 0)),
                  pl.BlockSpec((1, D), lambda i: (0, 0)),
                  pl.BlockSpec((1, D), lambda i: (0, 0))],
        out_specs=pl.BlockSpec((256, D), lambda i: (i, 0)),
        out_shape=jax.ShapeDtypeStruct((PAD_N, D), _F32),
    )(agg2, rn_col, Wv, Wo, bo.reshape(1, D), gamma.reshape(1, D), beta.reshape(1, D))


def kernel(query_node, key_edge, value_edge, edge_index, Wq, Wk, Wv, Wo, bo, gamma, beta):
    tgt = edge_index[1]
    p = _compute_p(query_node, Wq, Wk)
    e = _sc_edge(p, key_edge, tgt)
    agg2, sums2 = _sc_scatter(e, tgt, value_edge)
    rn = _compute_rn(sums2)
    wts = _sc_weights(e, tgt, rn.reshape(PAD_N))
    out = _finalize(agg2, rn.reshape(PAD_N, 1), Wv, Wo, bo, gamma, beta)
    return out[:N], wts


# speedup vs baseline: 9.6585x; 1.2617x over previous
import jax
import jax.numpy as jnp
from jax import lax
from jax.experimental import pallas as pl
from jax.experimental.pallas import tpu as pltpu
from jax.experimental.pallas import tpu_sc as plsc

N = 10000
E = 320000
D = 128
NC = 2
NS = 16
NW = NC * NS
EP = E // NW
C = 128
FC = EP // C
TAIL = EP - FC * C
PAD_N = 10240
RPS = PAD_N // NS
RPW = PAD_N // NW
SCALE = D ** -0.5

_F32 = jnp.float32
_I32 = jnp.int32


def _mesh():
    return plsc.VectorSubcoreMesh(
        core_axis_name="c", subcore_axis_name="s",
        num_cores=NC, num_subcores=NS)


_DNUMS = lax.GatherDimensionNumbers(
    offset_dims=(), collapsed_slice_dims=(0,), start_index_map=(0,))


def _perm(v, idx):
    return lax.gather(v, idx[:, None], _DNUMS, (1,),
                      mode=lax.GatherScatterMode.PROMISE_IN_BOUNDS)


def _hsum(v, lane):
    for sh in (8, 4, 2, 1):
        v = v + _perm(v, lane ^ sh)
    return v


def _hmax(v, lane):
    for sh in (8, 4, 2, 1):
        v = jnp.maximum(v, _perm(v, lane ^ sh))
    return v


def _dot(a, b, dims):
    return lax.dot_general(
        a, b, (dims, ((), ())),
        precision=lax.Precision.HIGHEST,
        preferred_element_type=_F32)


def _p_body(x_ref, wq_ref, wk_ref, o_ref):
    t = _dot(x_ref[...], wq_ref[...], ((1,), (0,)))
    o_ref[...] = _dot(t, wk_ref[...], ((1,), (1,))) * SCALE


def _compute_p(query, Wq, Wk):
    return pl.pallas_call(
        _p_body,
        grid=(25,),
        in_specs=[pl.BlockSpec((400, D), lambda i: (i, 0)),
                  pl.BlockSpec((D, D), lambda i: (0, 0)),
                  pl.BlockSpec((D, D), lambda i: (0, 0))],
        out_specs=pl.BlockSpec((400, D), lambda i: (i, 0)),
        out_shape=jax.ShapeDtypeStruct((N, D), _F32),
    )(query, Wq, Wk)


def _sc_edge_body(p_hbm, key_hbm, tgt_hbm, e_hbm,
                  idx_v, pg_v, key_v, e_v,
                  idx_t, pg_t, key_t, e_t,
                  sem_i, sem_k, sem_g):
    cid = lax.axis_index("c")
    sid = lax.axis_index("s")
    w = cid * NS + sid
    ebase = w * EP
    lane = lax.iota(_I32, 16)

    def run_chunk(base, idxr, pgr, keyr, er, n_edges):
        ci = pltpu.async_copy(tgt_hbm.at[pl.ds(base, n_edges)], idxr, sem_i)
        ck = pltpu.async_copy(key_hbm.at[pl.ds(base, n_edges)], keyr, sem_k)
        ci.wait()
        cg = pltpu.async_copy(p_hbm.at[idxr], pgr, sem_g)
        ck.wait()
        cg.wait()

        def sub(sb, svec):
            for c in range(4):
                i = sb * 4 + c
                acc = pgr[i, pl.ds(0, 16)] * keyr[i, pl.ds(0, 16)]
                for j in range(1, 8):
                    acc = acc + pgr[i, pl.ds(16 * j, 16)] * keyr[i, pl.ds(16 * j, 16)]
                svec = jnp.where(lane == (i & 15), _hsum(acc, lane), svec)

            @pl.when((sb & 3) == 3)
            def _store():
                er[pl.ds((sb >> 2) * 16, 16)] = jnp.exp(jnp.minimum(svec, 60.0))
            return jnp.where((sb & 3) == 3, jnp.zeros((16,), _F32), svec)

        lax.fori_loop(0, n_edges // 4, sub, jnp.zeros((16,), _F32))
        pltpu.sync_copy(er, e_hbm.at[pl.ds(base, n_edges)])

    def chunk(ci_, _):
        run_chunk(ebase + ci_ * C, idx_v, pg_v, key_v, e_v, C)
        return 0
    lax.fori_loop(0, FC, chunk, 0)
    run_chunk(ebase + FC * C, idx_t, pg_t, key_t, e_t, TAIL)


def _sc_edge(p, key_edge, tgt):
    f = pl.kernel(
        _sc_edge_body,
        out_type=jax.ShapeDtypeStruct((E,), _F32),
        mesh=_mesh(),
        scratch_types=[
            pltpu.VMEM((C,), _I32),
            pltpu.VMEM((C, D), _F32),
            pltpu.VMEM((C, D), _F32),
            pltpu.VMEM((C,), _F32),
            pltpu.VMEM((TAIL,), _I32),
            pltpu.VMEM((TAIL, D), _F32),
            pltpu.VMEM((TAIL, D), _F32),
            pltpu.VMEM((TAIL,), _F32),
            pltpu.SemaphoreType.DMA,
            pltpu.SemaphoreType.DMA,
            pltpu.SemaphoreType.DMA,
        ])
    return f(p, key_edge, tgt)


def _sc_scatter_body(e_hbm, tgt_hbm, val_hbm,
                     agg_hbm, sums_hbm,
                     idx_v, e_v, val_v, idx_t, e_t, val_t,
                     zrow_v, zsum_v,
                     sem_i, sem_e, sem_v, sem_s1, sem_s2,
                     agg_sh, sums_sh):
    cid = lax.axis_index("c")
    sid = lax.axis_index("s")
    w = cid * NS + sid
    ebase = w * EP
    lane = lax.iota(_I32, 16)

    def zr(r, _):
        for j in range(8):
            zrow_v[r, pl.ds(16 * j, 16)] = jnp.zeros((16,), _F32)
        return 0
    lax.fori_loop(0, 40, zr, 0)

    def zs(i, _):
        zsum_v[pl.ds(16 * i, 16)] = jnp.zeros((16,), _F32)
        return 0
    lax.fori_loop(0, RPS // 16, zs, 0)

    for k2 in range(RPS // 40):
        pltpu.sync_copy(zrow_v, agg_sh.at[pl.ds(sid * RPS + k2 * 40, 40)])
    pltpu.sync_copy(zsum_v, sums_sh.at[pl.ds(sid * RPS, RPS)])
    plsc.subcore_barrier()

    def run_chunk(base, idxr, er, vr, n_edges):
        c1 = pltpu.async_copy(tgt_hbm.at[pl.ds(base, n_edges)], idxr, sem_i)
        c2 = pltpu.async_copy(e_hbm.at[pl.ds(base, n_edges)], er, sem_e)
        c3 = pltpu.async_copy(val_hbm.at[pl.ds(base, n_edges)], vr, sem_v)
        c1.wait()
        c2.wait()
        c3.wait()

        def grp(g, _):
            ev16 = er[pl.ds(g * 16, 16)]
            for u in range(16):
                i = g * 16 + u
                wb = _perm(ev16, jnp.full((16,), u, _I32))
                for j in range(8):
                    slj = pl.ds(16 * j, 16)
                    vr[i, slj] = vr[i, slj] * wb
            return 0
        lax.fori_loop(0, n_edges // 16, grp, 0)

        cs1 = pltpu.async_copy(vr, agg_sh.at[idxr], sem_s1, add=True)
        cs2 = pltpu.async_copy(er, sums_sh.at[idxr], sem_s2, add=True)
        cs1.wait()
        cs2.wait()

    def chunk(ci, _):
        run_chunk(ebase + ci * C, idx_v, e_v, val_v, C)
        return 0
    lax.fori_loop(0, FC, chunk, 0)
    run_chunk(ebase + FC * C, idx_t, e_t, val_t, TAIL)

    plsc.subcore_barrier()
    pltpu.sync_copy(agg_sh.at[pl.ds(sid * RPS, RPS)],
                    agg_hbm.at[cid, pl.ds(sid * RPS, RPS)])
    pltpu.sync_copy(sums_sh.at[pl.ds(sid * RPS, RPS)],
                    sums_hbm.at[cid, pl.ds(sid * RPS, RPS)])


def _sc_scatter(e, tgt, value_edge):
    f = pl.kernel(
        _sc_scatter_body,
        out_type=(jax.ShapeDtypeStruct((NC, PAD_N, D), _F32),
                  jax.ShapeDtypeStruct((NC, PAD_N), _F32)),
        mesh=_mesh(),
        scratch_types=[
            pltpu.VMEM((C,), _I32),
            pltpu.VMEM((C,), _F32),
            pltpu.VMEM((C, D), _F32),
            pltpu.VMEM((TAIL,), _I32),
            pltpu.VMEM((TAIL,), _F32),
            pltpu.VMEM((TAIL, D), _F32),
            pltpu.VMEM((40, D), _F32),
            pltpu.VMEM((RPS,), _F32),
            pltpu.SemaphoreType.DMA,
            pltpu.SemaphoreType.DMA,
            pltpu.SemaphoreType.DMA,
            pltpu.SemaphoreType.DMA,
            pltpu.SemaphoreType.DMA,
            pltpu.VMEM_SHARED((PAD_N, D), _F32),
            pltpu.VMEM_SHARED((PAD_N,), _F32),
        ])
    return f(e, tgt, value_edge)


def _rn_body(s_ref, o_ref):
    o_ref[...] = 1.0 / (s_ref[0] + s_ref[1] + 1e-12)


def _compute_rn(sums2):
    return pl.pallas_call(
        _rn_body,
        out_shape=jax.ShapeDtypeStruct((PAD_N // D, D), _F32),
    )(sums2.reshape(NC, PAD_N // D, D))


def _sc_w_body(e_hbm, tgt_hbm, rn_hbm, w_hbm,
               idx_v, e_v, rg_v, w_v, idx_t, e_t, rg_t, w_t):
    cid = lax.axis_index("c")
    sid = lax.axis_index("s")
    w = cid * NS + sid
    ebase = w * EP

    def run_chunk(base, idxr, er, rgr, wr, n_edges):
        pltpu.sync_copy(tgt_hbm.at[pl.ds(base, n_edges)], idxr)
        pltpu.sync_copy(e_hbm.at[pl.ds(base, n_edges)], er)
        pltpu.sync_copy(rn_hbm.at[idxr], rgr)

        def grp(g, _):
            sl = pl.ds(g * 16, 16)
            wr[sl] = er[sl] * rgr[sl]
            return 0
        lax.fori_loop(0, n_edges // 16, grp, 0)
        pltpu.sync_copy(wr, w_hbm.at[pl.ds(base, n_edges)])

    def chunk(ci, _):
        run_chunk(ebase + ci * C, idx_v, e_v, rg_v, w_v, C)
        return 0
    lax.fori_loop(0, FC, chunk, 0)
    run_chunk(ebase + FC * C, idx_t, e_t, rg_t, w_t, TAIL)


def _sc_weights(e, tgt, rn_flat):
    f = pl.kernel(
        _sc_w_body,
        out_type=jax.ShapeDtypeStruct((E,), _F32),
        mesh=_mesh(),
        scratch_types=[
            pltpu.VMEM((C,), _I32),
            pltpu.VMEM((C,), _F32),
            pltpu.VMEM((C,), _F32),
            pltpu.VMEM((C,), _F32),
            pltpu.VMEM((TAIL,), _I32),
            pltpu.VMEM((TAIL,), _F32),
            pltpu.VMEM((TAIL,), _F32),
            pltpu.VMEM((TAIL,), _F32),
        ])
    return f(e, tgt, rn_flat)


def _out_body(a_ref, rn_ref, wv_ref, wo_ref, bo_ref, g_ref, b_ref, o_ref):
    a_n = (a_ref[0] + a_ref[1]) * rn_ref[...]
    t = _dot(a_n, wv_ref[...], ((1,), (0,)))
    proj = _dot(t, wo_ref[...], ((1,), (0,))) + bo_ref[...]
    mu = jnp.mean(proj, axis=1, keepdims=True)
    cen = proj - mu
    var = jnp.mean(cen * cen, axis=1, keepdims=True)
    o_ref[...] = cen * lax.rsqrt(var + 1e-5) * g_ref[...] + b_ref[...]


def _finalize(agg2, rn_col, Wv, Wo, bo, gamma, beta):
    return pl.pallas_call(
        _out_body,
        grid=(PAD_N // 256,),
        in_specs=[pl.BlockSpec((NC, 256, D), lambda i: (0, i, 0)),
                  pl.BlockSpec((256, 1), lambda i: (i, 0)),
                  pl.BlockSpec((D, D), lambda i: (0, 0)),
                  pl.BlockSpec((D, D), lambda i: (0, 0)),
                  pl.BlockSpec((1, D), lambda i: (0, 0)),
                  pl.BlockSpec((1, D), lambda i: (0, 0)),
                  pl.BlockSpec((1, D), lambda i: (0, 0))],
        out_specs=pl.BlockSpec((256, D), lambda i: (i, 0)),
        out_shape=jax.ShapeDtypeStruct((PAD_N, D), _F32),
    )(agg2, rn_col, Wv, Wo, bo.reshape(1, D), gamma.reshape(1, D), beta.reshape(1, D))


def kernel(query_node, key_edge, value_edge, edge_index, Wq, Wk, Wv, Wo, bo, gamma, beta):
    tgt = edge_index[1]
    p = _compute_p(query_node, Wq, Wk)
    e = _sc_edge(p, key_edge, tgt)
    agg2, sums2 = _sc_scatter(e, tgt, value_edge)
    rn = _compute_rn(sums2)
    wts = _sc_weights(e, tgt, rn.reshape(PAD_N))
    out = _finalize(agg2, rn.reshape(PAD_N, 1), Wv, Wo, bo, gamma, beta)
    return out[:N], wts


# speedup vs baseline: 14.9642x; 1.5493x over previous
import jax
import jax.numpy as jnp
from jax import lax
from jax.experimental import pallas as pl
from jax.experimental.pallas import tpu as pltpu
from jax.experimental.pallas import tpu_sc as plsc

N = 10000
E = 320000
D = 128
NC = 2
NS = 16
NW = NC * NS
EP = E // NW
C = 128
FC = EP // C
TAIL = EP - FC * C
PAD_N = 10240
RPS = PAD_N // NS
RPW = PAD_N // NW
SCALE = D ** -0.5

_F32 = jnp.float32
_I32 = jnp.int32


def _mesh():
    return plsc.VectorSubcoreMesh(
        core_axis_name="c", subcore_axis_name="s",
        num_cores=NC, num_subcores=NS)


_DNUMS = lax.GatherDimensionNumbers(
    offset_dims=(), collapsed_slice_dims=(0,), start_index_map=(0,))


def _perm(v, idx):
    return lax.gather(v, idx[:, None], _DNUMS, (1,),
                      mode=lax.GatherScatterMode.PROMISE_IN_BOUNDS)


def _hsum(v, lane):
    for sh in (8, 4, 2, 1):
        v = v + _perm(v, lane ^ sh)
    return v


def _hmax(v, lane):
    for sh in (8, 4, 2, 1):
        v = jnp.maximum(v, _perm(v, lane ^ sh))
    return v


def _dot(a, b, dims):
    return lax.dot_general(
        a, b, (dims, ((), ())),
        precision=lax.Precision.HIGHEST,
        preferred_element_type=_F32)


def _p_body(x_ref, wq_ref, wk_ref, o_ref):
    t = _dot(x_ref[...], wq_ref[...], ((1,), (0,)))
    o_ref[...] = _dot(t, wk_ref[...], ((1,), (1,))) * SCALE


def _compute_p(query, Wq, Wk):
    return pl.pallas_call(
        _p_body,
        grid=(25,),
        in_specs=[pl.BlockSpec((400, D), lambda i: (i, 0)),
                  pl.BlockSpec((D, D), lambda i: (0, 0)),
                  pl.BlockSpec((D, D), lambda i: (0, 0))],
        out_specs=pl.BlockSpec((400, D), lambda i: (i, 0)),
        out_shape=jax.ShapeDtypeStruct((N, D), _F32),
    )(query, Wq, Wk)


_HALF = FC // 2


def _sc_edge_body(p_hbm, key_hbm, tgt_hbm, e_hbm,
                  idx0, idx1, pg0, pg1, key0, key1, e0, e1,
                  idx_t, pg_t, key_t, e_t,
                  si0, si1, sk0, sk1, sg0, sg1, se0, se1):
    cid = lax.axis_index("c")
    sid = lax.axis_index("s")
    w = cid * NS + sid
    ebase = w * EP
    lane = lax.iota(_I32, 16)
    idx_ = (idx0, idx1)
    pg_ = (pg0, pg1)
    key_ = (key0, key1)
    e_ = (e0, e1)
    si_ = (si0, si1)
    sk_ = (sk0, sk1)
    sg_ = (sg0, sg1)
    se_ = (se0, se1)

    def issue_loads(ci, b):
        base = ebase + ci * C
        pltpu.async_copy(tgt_hbm.at[pl.ds(base, C)], idx_[b], si_[b])
        pltpu.async_copy(key_hbm.at[pl.ds(base, C)], key_[b], sk_[b])

    def wait_idx(b):
        pltpu.make_async_copy(tgt_hbm.at[pl.ds(0, C)], idx_[b], si_[b]).wait()

    def issue_gather(b):
        pltpu.async_copy(p_hbm.at[idx_[b]], pg_[b], sg_[b])

    def wait_key_gather(b):
        pltpu.make_async_copy(key_hbm.at[pl.ds(0, C)], key_[b], sk_[b]).wait()
        pltpu.make_async_copy(p_hbm.at[idx_[b]], pg_[b], sg_[b]).wait()

    def issue_estore(ci, b):
        pltpu.async_copy(e_[b], e_hbm.at[pl.ds(ebase + ci * C, C)], se_[b])

    def wait_estore(b):
        pltpu.make_async_copy(e_[b], e_hbm.at[pl.ds(0, C)], se_[b]).wait()

    def compute(pgr, keyr, er, n_edges):
        def sub(sb, svec):
            for c in range(4):
                i = sb * 4 + c
                acc = pgr[i, pl.ds(0, 16)] * keyr[i, pl.ds(0, 16)]
                for j in range(1, 8):
                    acc = acc + pgr[i, pl.ds(16 * j, 16)] * keyr[i, pl.ds(16 * j, 16)]
                svec = jnp.where(lane == (i & 15), _hsum(acc, lane), svec)

            @pl.when((sb & 3) == 3)
            def _store():
                er[pl.ds((sb >> 2) * 16, 16)] = jnp.exp(jnp.minimum(svec, 60.0))
            return jnp.where((sb & 3) == 3, jnp.zeros((16,), _F32), svec)

        lax.fori_loop(0, n_edges // 4, sub, jnp.zeros((16,), _F32))

    issue_loads(0, 0)
    wait_idx(0)
    issue_gather(0)
    issue_loads(1, 1)

    def body(t2, _):
        a = 2 * t2
        not_last = t2 < _HALF - 1

        wait_key_gather(0)
        wait_idx(1)
        issue_gather(1)

        @pl.when(t2 > 0)
        def _d0():
            wait_estore(0)
        compute(pg0, key0, e0, C)
        issue_estore(a, 0)

        @pl.when(not_last)
        def _p0():
            issue_loads(a + 2, 0)

        wait_key_gather(1)

        @pl.when(not_last)
        def _g0():
            wait_idx(0)
            issue_gather(0)

        @pl.when(t2 > 0)
        def _d1():
            wait_estore(1)
        compute(pg1, key1, e1, C)
        issue_estore(a + 1, 1)

        @pl.when(not_last)
        def _p1():
            issue_loads(a + 3, 1)
        return 0

    lax.fori_loop(0, _HALF, body, 0)
    wait_estore(0)
    wait_estore(1)

    tbase = ebase + FC * C
    pltpu.sync_copy(tgt_hbm.at[pl.ds(tbase, TAIL)], idx_t)
    pltpu.sync_copy(key_hbm.at[pl.ds(tbase, TAIL)], key_t)
    pltpu.sync_copy(p_hbm.at[idx_t], pg_t)
    compute(pg_t, key_t, e_t, TAIL)
    pltpu.sync_copy(e_t, e_hbm.at[pl.ds(tbase, TAIL)])


def _sc_edge(p, key_edge, tgt):
    f = pl.kernel(
        _sc_edge_body,
        out_type=jax.ShapeDtypeStruct((E,), _F32),
        mesh=_mesh(),
        scratch_types=[
            pltpu.VMEM((C,), _I32),
            pltpu.VMEM((C,), _I32),
            pltpu.VMEM((C, D), _F32),
            pltpu.VMEM((C, D), _F32),
            pltpu.VMEM((C, D), _F32),
            pltpu.VMEM((C, D), _F32),
            pltpu.VMEM((C,), _F32),
            pltpu.VMEM((C,), _F32),
            pltpu.VMEM((TAIL,), _I32),
            pltpu.VMEM((TAIL, D), _F32),
            pltpu.VMEM((TAIL, D), _F32),
            pltpu.VMEM((TAIL,), _F32),
        ] + [pltpu.SemaphoreType.DMA] * 8)
    return f(p, key_edge, tgt)


def _sc_scatter_body(e_hbm, tgt_hbm, val_hbm,
                     agg_hbm, sums_hbm,
                     idx0, idx1, e0, e1, val0, val1,
                     idx_t, e_t, val_t,
                     zrow_v, zsum_v,
                     si0, si1, se0, se1, sv0, sv1, sa0, sa1, ss0, ss1,
                     agg_sh, sums_sh):
    cid = lax.axis_index("c")
    sid = lax.axis_index("s")
    w = cid * NS + sid
    ebase = w * EP
    lane = lax.iota(_I32, 16)
    idx_ = (idx0, idx1)
    e_ = (e0, e1)
    val_ = (val0, val1)
    si_ = (si0, si1)
    se_ = (se0, se1)
    sv_ = (sv0, sv1)
    sa_ = (sa0, sa1)
    ss_ = (ss0, ss1)

    def zr(r, _):
        for j in range(8):
            zrow_v[r, pl.ds(16 * j, 16)] = jnp.zeros((16,), _F32)
        return 0
    lax.fori_loop(0, 40, zr, 0)

    def zs(i, _):
        zsum_v[pl.ds(16 * i, 16)] = jnp.zeros((16,), _F32)
        return 0
    lax.fori_loop(0, RPS // 16, zs, 0)

    for k2 in range(RPS // 40):
        pltpu.sync_copy(zrow_v, agg_sh.at[pl.ds(sid * RPS + k2 * 40, 40)])
    pltpu.sync_copy(zsum_v, sums_sh.at[pl.ds(sid * RPS, RPS)])
    plsc.subcore_barrier()

    def issue_loads(ci, b):
        base = ebase + ci * C
        pltpu.async_copy(tgt_hbm.at[pl.ds(base, C)], idx_[b], si_[b])
        pltpu.async_copy(e_hbm.at[pl.ds(base, C)], e_[b], se_[b])
        pltpu.async_copy(val_hbm.at[pl.ds(base, C)], val_[b], sv_[b])

    def wait_loads(b):
        pltpu.make_async_copy(tgt_hbm.at[pl.ds(0, C)], idx_[b], si_[b]).wait()
        pltpu.make_async_copy(e_hbm.at[pl.ds(0, C)], e_[b], se_[b]).wait()
        pltpu.make_async_copy(val_hbm.at[pl.ds(0, C)], val_[b], sv_[b]).wait()

    def scale(er, vr, n_edges):
        def grp(g, _):
            ev16 = er[pl.ds(g * 16, 16)]
            for u in range(16):
                i = g * 16 + u
                wb = _perm(ev16, jnp.full((16,), u, _I32))
                for j in range(8):
                    slj = pl.ds(16 * j, 16)
                    vr[i, slj] = vr[i, slj] * wb
            return 0
        lax.fori_loop(0, n_edges // 16, grp, 0)

    def issue_scatters(b):
        pltpu.async_copy(val_[b], agg_sh.at[idx_[b]], sa_[b], add=True)
        pltpu.async_copy(e_[b], sums_sh.at[idx_[b]], ss_[b], add=True)

    def wait_scatters(b):
        pltpu.make_async_copy(val_[b], agg_sh.at[idx_[b]], sa_[b]).wait()
        pltpu.make_async_copy(e_[b], sums_sh.at[idx_[b]], ss_[b]).wait()

    issue_loads(0, 0)
    issue_loads(1, 1)

    def body(t2, _):
        a = 2 * t2
        not_last = t2 < _HALF - 1

        wait_loads(0)
        scale(e0, val0, C)
        issue_scatters(0)

        wait_loads(1)
        scale(e1, val1, C)
        issue_scatters(1)

        @pl.when(not_last)
        def _p():
            wait_scatters(0)
            issue_loads(a + 2, 0)
            wait_scatters(1)
            issue_loads(a + 3, 1)
        return 0

    lax.fori_loop(0, _HALF, body, 0)
    wait_scatters(0)
    wait_scatters(1)

    tbase = ebase + FC * C
    pltpu.sync_copy(tgt_hbm.at[pl.ds(tbase, TAIL)], idx_t)
    pltpu.sync_copy(e_hbm.at[pl.ds(tbase, TAIL)], e_t)
    pltpu.sync_copy(val_hbm.at[pl.ds(tbase, TAIL)], val_t)
    scale(e_t, val_t, TAIL)
    pltpu.sync_copy(val_t, agg_sh.at[idx_t], add=True)
    pltpu.sync_copy(e_t, sums_sh.at[idx_t], add=True)

    plsc.subcore_barrier()
    pltpu.sync_copy(agg_sh.at[pl.ds(sid * RPS, RPS)],
                    agg_hbm.at[cid, pl.ds(sid * RPS, RPS)])
    pltpu.sync_copy(sums_sh.at[pl.ds(sid * RPS, RPS)],
                    sums_hbm.at[cid, pl.ds(sid * RPS, RPS)])


def _sc_scatter(e, tgt, value_edge):
    f = pl.kernel(
        _sc_scatter_body,
        out_type=(jax.ShapeDtypeStruct((NC, PAD_N, D), _F32),
                  jax.ShapeDtypeStruct((NC, PAD_N), _F32)),
        mesh=_mesh(),
        scratch_types=[
            pltpu.VMEM((C,), _I32),
            pltpu.VMEM((C,), _I32),
            pltpu.VMEM((C,), _F32),
            pltpu.VMEM((C,), _F32),
            pltpu.VMEM((C, D), _F32),
            pltpu.VMEM((C, D), _F32),
            pltpu.VMEM((TAIL,), _I32),
            pltpu.VMEM((TAIL,), _F32),
            pltpu.VMEM((TAIL, D), _F32),
            pltpu.VMEM((40, D), _F32),
            pltpu.VMEM((RPS,), _F32),
        ] + [pltpu.SemaphoreType.DMA] * 10 + [
            pltpu.VMEM_SHARED((PAD_N, D), _F32),
            pltpu.VMEM_SHARED((PAD_N,), _F32),
        ])
    return f(e, tgt, value_edge)


def _rn_body(s_ref, o_ref):
    o_ref[...] = 1.0 / (s_ref[0] + s_ref[1] + 1e-12)


def _compute_rn(sums2):
    return pl.pallas_call(
        _rn_body,
        out_shape=jax.ShapeDtypeStruct((PAD_N // D, D), _F32),
    )(sums2.reshape(NC, PAD_N // D, D))


def _sc_w_body(e_hbm, tgt_hbm, rn_hbm, w_hbm,
               idx0, idx1, e0, e1, rg0, rg1, w0, w1,
               idx_t, e_t, rg_t, w_t,
               si0, si1, se0, se1, sg0, sg1, sw0, sw1):
    cid = lax.axis_index("c")
    sid = lax.axis_index("s")
    w = cid * NS + sid
    ebase = w * EP
    idx_ = (idx0, idx1)
    e_ = (e0, e1)
    rg_ = (rg0, rg1)
    w_ = (w0, w1)
    si_ = (si0, si1)
    se_ = (se0, se1)
    sg_ = (sg0, sg1)
    sw_ = (sw0, sw1)

    def issue_loads(ci, b):
        base = ebase + ci * C
        pltpu.async_copy(tgt_hbm.at[pl.ds(base, C)], idx_[b], si_[b])
        pltpu.async_copy(e_hbm.at[pl.ds(base, C)], e_[b], se_[b])

    def wait_idx(b):
        pltpu.make_async_copy(tgt_hbm.at[pl.ds(0, C)], idx_[b], si_[b]).wait()

    def issue_gather(b):
        pltpu.async_copy(rn_hbm.at[idx_[b]], rg_[b], sg_[b])

    def wait_e_gather(b):
        pltpu.make_async_copy(e_hbm.at[pl.ds(0, C)], e_[b], se_[b]).wait()
        pltpu.make_async_copy(rn_hbm.at[idx_[b]], rg_[b], sg_[b]).wait()

    def issue_wstore(ci, b):
        pltpu.async_copy(w_[b], w_hbm.at[pl.ds(ebase + ci * C, C)], sw_[b])

    def wait_wstore(b):
        pltpu.make_async_copy(w_[b], w_hbm.at[pl.ds(0, C)], sw_[b]).wait()

    def compute(er, rgr, wr, n_edges):
        def grp(g, _):
            sl = pl.ds(g * 16, 16)
            wr[sl] = er[sl] * rgr[sl]
            return 0
        lax.fori_loop(0, n_edges // 16, grp, 0)

    issue_loads(0, 0)
    wait_idx(0)
    issue_gather(0)
    issue_loads(1, 1)

    def body(t2, _):
        a = 2 * t2
        not_last = t2 < _HALF - 1

        wait_e_gather(0)
        wait_idx(1)
        issue_gather(1)

        @pl.when(t2 > 0)
        def _d0():
            wait_wstore(0)
        compute(e0, rg0, w0, C)
        issue_wstore(a, 0)

        @pl.when(not_last)
        def _p0():
            issue_loads(a + 2, 0)

        wait_e_gather(1)

        @pl.when(not_last)
        def _g0():
            wait_idx(0)
            issue_gather(0)

        @pl.when(t2 > 0)
        def _d1():
            wait_wstore(1)
        compute(e1, rg1, w1, C)
        issue_wstore(a + 1, 1)

        @pl.when(not_last)
        def _p1():
            issue_loads(a + 3, 1)
        return 0

    lax.fori_loop(0, _HALF, body, 0)
    wait_wstore(0)
    wait_wstore(1)

    tbase = ebase + FC * C
    pltpu.sync_copy(tgt_hbm.at[pl.ds(tbase, TAIL)], idx_t)
    pltpu.sync_copy(e_hbm.at[pl.ds(tbase, TAIL)], e_t)
    pltpu.sync_copy(rn_hbm.at[idx_t], rg_t)
    compute(e_t, rg_t, w_t, TAIL)
    pltpu.sync_copy(w_t, w_hbm.at[pl.ds(tbase, TAIL)])


def _sc_weights(e, tgt, rn_flat):
    f = pl.kernel(
        _sc_w_body,
        out_type=jax.ShapeDtypeStruct((E,), _F32),
        mesh=_mesh(),
        scratch_types=[
            pltpu.VMEM((C,), _I32),
            pltpu.VMEM((C,), _I32),
            pltpu.VMEM((C,), _F32),
            pltpu.VMEM((C,), _F32),
            pltpu.VMEM((C,), _F32),
            pltpu.VMEM((C,), _F32),
            pltpu.VMEM((C,), _F32),
            pltpu.VMEM((C,), _F32),
            pltpu.VMEM((TAIL,), _I32),
            pltpu.VMEM((TAIL,), _F32),
            pltpu.VMEM((TAIL,), _F32),
            pltpu.VMEM((TAIL,), _F32),
        ] + [pltpu.SemaphoreType.DMA] * 8)
    return f(e, tgt, rn_flat)


def _out_body(a_ref, rn_ref, wv_ref, wo_ref, bo_ref, g_ref, b_ref, o_ref):
    a_n = (a_ref[0] + a_ref[1]) * rn_ref[...]
    t = _dot(a_n, wv_ref[...], ((1,), (0,)))
    proj = _dot(t, wo_ref[...], ((1,), (0,))) + bo_ref[...]
    mu = jnp.mean(proj, axis=1, keepdims=True)
    cen = proj - mu
    var = jnp.mean(cen * cen, axis=1, keepdims=True)
    o_ref[...] = cen * lax.rsqrt(var + 1e-5) * g_ref[...] + b_ref[...]


def _finalize(agg2, rn_col, Wv, Wo, bo, gamma, beta):
    return pl.pallas_call(
        _out_body,
        grid=(PAD_N // 256,),
        in_specs=[pl.BlockSpec((NC, 256, D), lambda i: (0, i, 0)),
                  pl.BlockSpec((256, 1), lambda i: (i, 0)),
                  pl.BlockSpec((D, D), lambda i: (0, 0)),
                  pl.BlockSpec((D, D), lambda i: (0, 0)),
                  pl.BlockSpec((1, D), lambda i: (0, 0)),
                  pl.BlockSpec((1, D), lambda i: (0, 0)),
                  pl.BlockSpec((1, D), lambda i: (0, 0))],
        out_specs=pl.BlockSpec((256, D), lambda i: (i, 0)),
        out_shape=jax.ShapeDtypeStruct((PAD_N, D), _F32),
    )(agg2, rn_col, Wv, Wo, bo.reshape(1, D), gamma.reshape(1, D), beta.reshape(1, D))


def kernel(query_node, key_edge, value_edge, edge_index, Wq, Wk, Wv, Wo, bo, gamma, beta):
    tgt = edge_index[1]
    p = _compute_p(query_node, Wq, Wk)
    e = _sc_edge(p, key_edge, tgt)
    agg2, sums2 = _sc_scatter(e, tgt, value_edge)
    rn = _compute_rn(sums2)
    wts = _sc_weights(e, tgt, rn.reshape(PAD_N))
    out = _finalize(agg2, rn.reshape(PAD_N, 1), Wv, Wo, bo, gamma, beta)
    return out[:N], wts


# speedup vs baseline: 15.9135x; 1.0634x over previous
import jax
import jax.numpy as jnp
from jax import lax
from jax.experimental import pallas as pl
from jax.experimental.pallas import tpu as pltpu
from jax.experimental.pallas import tpu_sc as plsc

N = 10000
E = 320000
D = 128
NC = 2
NS = 16
NW = NC * NS
EP = E // NW
C = 128
FC = EP // C
TAIL = EP - FC * C
PAD_N = 10240
RPS = PAD_N // NS
RPW = PAD_N // NW
SCALE = D ** -0.5

_F32 = jnp.float32
_I32 = jnp.int32


def _mesh():
    return plsc.VectorSubcoreMesh(
        core_axis_name="c", subcore_axis_name="s",
        num_cores=NC, num_subcores=NS)


_DNUMS = lax.GatherDimensionNumbers(
    offset_dims=(), collapsed_slice_dims=(0,), start_index_map=(0,))


def _perm(v, idx):
    return lax.gather(v, idx[:, None], _DNUMS, (1,),
                      mode=lax.GatherScatterMode.PROMISE_IN_BOUNDS)


def _hsum(v, lane):
    for sh in (8, 4, 2, 1):
        v = v + _perm(v, lane ^ sh)
    return v


def _hmax(v, lane):
    for sh in (8, 4, 2, 1):
        v = jnp.maximum(v, _perm(v, lane ^ sh))
    return v


def _dot(a, b, dims):
    return lax.dot_general(
        a, b, (dims, ((), ())),
        precision=lax.Precision.HIGHEST,
        preferred_element_type=_F32)


def _p_body(x_ref, wq_ref, wk_ref, o_ref):
    t = _dot(x_ref[...], wq_ref[...], ((1,), (0,)))
    o_ref[...] = _dot(t, wk_ref[...], ((1,), (1,))) * SCALE


def _compute_p(query, Wq, Wk):
    return pl.pallas_call(
        _p_body,
        grid=(25,),
        in_specs=[pl.BlockSpec((400, D), lambda i: (i, 0)),
                  pl.BlockSpec((D, D), lambda i: (0, 0)),
                  pl.BlockSpec((D, D), lambda i: (0, 0))],
        out_specs=pl.BlockSpec((400, D), lambda i: (i, 0)),
        out_shape=jax.ShapeDtypeStruct((N, D), _F32),
    )(query, Wq, Wk)


_HALF = FC // 2


def _sc_edge_body(p_hbm, key_hbm, tgt_hbm, e_hbm,
                  idx0, idx1, pg0, pg1, key0, key1, e0, e1,
                  idx_t, pg_t, key_t, e_t,
                  si0, si1, sk0, sk1, sg0, sg1, se0, se1):
    cid = lax.axis_index("c")
    sid = lax.axis_index("s")
    w = cid * NS + sid
    ebase = w * EP
    lane = lax.iota(_I32, 16)
    idx_ = (idx0, idx1)
    pg_ = (pg0, pg1)
    key_ = (key0, key1)
    e_ = (e0, e1)
    si_ = (si0, si1)
    sk_ = (sk0, sk1)
    sg_ = (sg0, sg1)
    se_ = (se0, se1)

    def issue_loads(ci, b):
        base = ebase + ci * C
        pltpu.async_copy(tgt_hbm.at[pl.ds(base, C)], idx_[b], si_[b])
        pltpu.async_copy(key_hbm.at[pl.ds(base, C)], key_[b], sk_[b])

    def wait_idx(b):
        pltpu.make_async_copy(tgt_hbm.at[pl.ds(0, C)], idx_[b], si_[b]).wait()

    def issue_gather(b):
        pltpu.async_copy(p_hbm.at[idx_[b]], pg_[b], sg_[b])

    def wait_key_gather(b):
        pltpu.make_async_copy(key_hbm.at[pl.ds(0, C)], key_[b], sk_[b]).wait()
        pltpu.make_async_copy(p_hbm.at[idx_[b]], pg_[b], sg_[b]).wait()

    def issue_estore(ci, b):
        pltpu.async_copy(e_[b], e_hbm.at[pl.ds(ebase + ci * C, C)], se_[b])

    def wait_estore(b):
        pltpu.make_async_copy(e_[b], e_hbm.at[pl.ds(0, C)], se_[b]).wait()

    def compute(pgr, keyr, er, n_edges):
        def sub(sb, svec):
            for c in range(4):
                i = sb * 4 + c
                acc = pgr[i, pl.ds(0, 16)] * keyr[i, pl.ds(0, 16)]
                for j in range(1, 8):
                    acc = acc + pgr[i, pl.ds(16 * j, 16)] * keyr[i, pl.ds(16 * j, 16)]
                svec = jnp.where(lane == (i & 15), _hsum(acc, lane), svec)

            @pl.when((sb & 3) == 3)
            def _store():
                er[pl.ds((sb >> 2) * 16, 16)] = jnp.exp(jnp.minimum(svec, 60.0))
            return jnp.where((sb & 3) == 3, jnp.zeros((16,), _F32), svec)

        lax.fori_loop(0, n_edges // 4, sub, jnp.zeros((16,), _F32))

    issue_loads(0, 0)
    wait_idx(0)
    issue_gather(0)
    issue_loads(1, 1)

    def body(t2, _):
        a = 2 * t2
        not_last = t2 < _HALF - 1

        wait_key_gather(0)
        wait_idx(1)
        issue_gather(1)

        @pl.when(t2 > 0)
        def _d0():
            wait_estore(0)
        compute(pg0, key0, e0, C)
        issue_estore(a, 0)

        @pl.when(not_last)
        def _p0():
            issue_loads(a + 2, 0)

        wait_key_gather(1)

        @pl.when(not_last)
        def _g0():
            wait_idx(0)
            issue_gather(0)

        @pl.when(t2 > 0)
        def _d1():
            wait_estore(1)
        compute(pg1, key1, e1, C)
        issue_estore(a + 1, 1)

        @pl.when(not_last)
        def _p1():
            issue_loads(a + 3, 1)
        return 0

    lax.fori_loop(0, _HALF, body, 0)
    wait_estore(0)
    wait_estore(1)

    tbase = ebase + FC * C
    pltpu.sync_copy(tgt_hbm.at[pl.ds(tbase, TAIL)], idx_t)
    pltpu.sync_copy(key_hbm.at[pl.ds(tbase, TAIL)], key_t)
    pltpu.sync_copy(p_hbm.at[idx_t], pg_t)
    compute(pg_t, key_t, e_t, TAIL)
    pltpu.sync_copy(e_t, e_hbm.at[pl.ds(tbase, TAIL)])


def _sc_edge(p, key_edge, tgt):
    f = pl.kernel(
        _sc_edge_body,
        out_type=jax.ShapeDtypeStruct((E,), _F32),
        mesh=_mesh(),
        scratch_types=[
            pltpu.VMEM((C,), _I32),
            pltpu.VMEM((C,), _I32),
            pltpu.VMEM((C, D), _F32),
            pltpu.VMEM((C, D), _F32),
            pltpu.VMEM((C, D), _F32),
            pltpu.VMEM((C, D), _F32),
            pltpu.VMEM((C,), _F32),
            pltpu.VMEM((C,), _F32),
            pltpu.VMEM((TAIL,), _I32),
            pltpu.VMEM((TAIL, D), _F32),
            pltpu.VMEM((TAIL, D), _F32),
            pltpu.VMEM((TAIL,), _F32),
        ] + [pltpu.SemaphoreType.DMA] * 8)
    return f(p, key_edge, tgt)


def _sc_scatter_body(e_hbm, tgt_hbm, val_hbm,
                     agg_hbm, sums_hbm,
                     idx0, idx1, e0, e1, val0, val1,
                     idx_t, e_t, val_t,
                     zrow_v, zsum_v,
                     si0, si1, se0, se1, sv0, sv1, sa0, sa1, ss0, ss1,
                     agg_sh, sums_sh):
    cid = lax.axis_index("c")
    sid = lax.axis_index("s")
    w = cid * NS + sid
    ebase = w * EP
    lane = lax.iota(_I32, 16)
    idx_ = (idx0, idx1)
    e_ = (e0, e1)
    val_ = (val0, val1)
    si_ = (si0, si1)
    se_ = (se0, se1)
    sv_ = (sv0, sv1)
    sa_ = (sa0, sa1)
    ss_ = (ss0, ss1)

    def zr(r, _):
        for j in range(8):
            zrow_v[r, pl.ds(16 * j, 16)] = jnp.zeros((16,), _F32)
        return 0
    lax.fori_loop(0, 40, zr, 0)

    def zs(i, _):
        zsum_v[pl.ds(16 * i, 16)] = jnp.zeros((16,), _F32)
        return 0
    lax.fori_loop(0, RPS // 16, zs, 0)

    for k2 in range(RPS // 40):
        pltpu.sync_copy(zrow_v, agg_sh.at[pl.ds(sid * RPS + k2 * 40, 40)])
    pltpu.sync_copy(zsum_v, sums_sh.at[pl.ds(sid * RPS, RPS)])
    plsc.subcore_barrier()

    def issue_loads(ci, b):
        base = ebase + ci * C
        pltpu.async_copy(tgt_hbm.at[pl.ds(base, C)], idx_[b], si_[b])
        pltpu.async_copy(e_hbm.at[pl.ds(base, C)], e_[b], se_[b])
        pltpu.async_copy(val_hbm.at[pl.ds(base, C)], val_[b], sv_[b])

    def wait_loads(b):
        pltpu.make_async_copy(tgt_hbm.at[pl.ds(0, C)], idx_[b], si_[b]).wait()
        pltpu.make_async_copy(e_hbm.at[pl.ds(0, C)], e_[b], se_[b]).wait()
        pltpu.make_async_copy(val_hbm.at[pl.ds(0, C)], val_[b], sv_[b]).wait()

    def scale(er, vr, n_edges):
        def grp(g, _):
            ev16 = er[pl.ds(g * 16, 16)]
            for u in range(16):
                i = g * 16 + u
                wb = _perm(ev16, jnp.full((16,), u, _I32))
                for j in range(8):
                    slj = pl.ds(16 * j, 16)
                    vr[i, slj] = vr[i, slj] * wb
            return 0
        lax.fori_loop(0, n_edges // 16, grp, 0)

    def issue_scatters(b):
        pltpu.async_copy(val_[b], agg_sh.at[idx_[b]], sa_[b], add=True)
        pltpu.async_copy(e_[b], sums_sh.at[idx_[b]], ss_[b], add=True)

    def wait_scatters(b):
        pltpu.make_async_copy(val_[b], agg_sh.at[idx_[b]], sa_[b]).wait()
        pltpu.make_async_copy(e_[b], sums_sh.at[idx_[b]], ss_[b]).wait()

    issue_loads(0, 0)
    issue_loads(1, 1)

    def body(t2, _):
        a = 2 * t2
        not_last = t2 < _HALF - 1

        wait_loads(0)
        scale(e0, val0, C)
        issue_scatters(0)

        wait_loads(1)
        scale(e1, val1, C)
        issue_scatters(1)

        @pl.when(not_last)
        def _p():
            wait_scatters(0)
            issue_loads(a + 2, 0)
            wait_scatters(1)
            issue_loads(a + 3, 1)
        return 0

    lax.fori_loop(0, _HALF, body, 0)
    wait_scatters(0)
    wait_scatters(1)

    tbase = ebase + FC * C
    pltpu.sync_copy(tgt_hbm.at[pl.ds(tbase, TAIL)], idx_t)
    pltpu.sync_copy(e_hbm.at[pl.ds(tbase, TAIL)], e_t)
    pltpu.sync_copy(val_hbm.at[pl.ds(tbase, TAIL)], val_t)
    scale(e_t, val_t, TAIL)
    pltpu.sync_copy(val_t, agg_sh.at[idx_t], add=True)
    pltpu.sync_copy(e_t, sums_sh.at[idx_t], add=True)

    plsc.subcore_barrier()
    pltpu.sync_copy(agg_sh.at[pl.ds(sid * RPS, RPS)],
                    agg_hbm.at[cid, pl.ds(sid * RPS, RPS)])
    pltpu.sync_copy(sums_sh.at[pl.ds(sid * RPS, RPS)],
                    sums_hbm.at[cid, pl.ds(sid * RPS, RPS)])


def _sc_scatter(e, tgt, value_edge):
    f = pl.kernel(
        _sc_scatter_body,
        out_type=(jax.ShapeDtypeStruct((NC, PAD_N, D), _F32),
                  jax.ShapeDtypeStruct((NC, PAD_N), _F32)),
        mesh=_mesh(),
        scratch_types=[
            pltpu.VMEM((C,), _I32),
            pltpu.VMEM((C,), _I32),
            pltpu.VMEM((C,), _F32),
            pltpu.VMEM((C,), _F32),
            pltpu.VMEM((C, D), _F32),
            pltpu.VMEM((C, D), _F32),
            pltpu.VMEM((TAIL,), _I32),
            pltpu.VMEM((TAIL,), _F32),
            pltpu.VMEM((TAIL, D), _F32),
            pltpu.VMEM((40, D), _F32),
            pltpu.VMEM((RPS,), _F32),
        ] + [pltpu.SemaphoreType.DMA] * 10 + [
            pltpu.VMEM_SHARED((PAD_N, D), _F32),
            pltpu.VMEM_SHARED((PAD_N,), _F32),
        ])
    return f(e, tgt, value_edge)


def _rn_body(s_ref, o_ref):
    o_ref[...] = 1.0 / (s_ref[0] + s_ref[1] + 1e-12)


def _compute_rn(sums2):
    return pl.pallas_call(
        _rn_body,
        out_shape=jax.ShapeDtypeStruct((PAD_N // D, D), _F32),
    )(sums2.reshape(NC, PAD_N // D, D))


def _sc_w_body(e_hbm, tgt_hbm, rn_hbm, w_hbm,
               idx0, idx1, e0, e1, rg0, rg1, w0, w1,
               idx_t, e_t, rg_t, w_t,
               si0, si1, se0, se1, sg0, sg1, sw0, sw1,
               rn_sh):
    cid = lax.axis_index("c")
    sid = lax.axis_index("s")
    w = cid * NS + sid
    ebase = w * EP
    idx_ = (idx0, idx1)
    e_ = (e0, e1)
    rg_ = (rg0, rg1)
    w_ = (w0, w1)
    si_ = (si0, si1)
    se_ = (se0, se1)
    sg_ = (sg0, sg1)
    sw_ = (sw0, sw1)

    @pl.when(sid == 0)
    def _stage():
        pltpu.sync_copy(rn_hbm, rn_sh)
    plsc.subcore_barrier()

    def issue_loads(ci, b):
        base = ebase + ci * C
        pltpu.async_copy(tgt_hbm.at[pl.ds(base, C)], idx_[b], si_[b])
        pltpu.async_copy(e_hbm.at[pl.ds(base, C)], e_[b], se_[b])

    def wait_idx(b):
        pltpu.make_async_copy(tgt_hbm.at[pl.ds(0, C)], idx_[b], si_[b]).wait()

    def issue_gather(b):
        pltpu.async_copy(rn_sh.at[idx_[b]], rg_[b], sg_[b])

    def wait_e_gather(b):
        pltpu.make_async_copy(e_hbm.at[pl.ds(0, C)], e_[b], se_[b]).wait()
        pltpu.make_async_copy(rn_sh.at[idx_[b]], rg_[b], sg_[b]).wait()

    def issue_wstore(ci, b):
        pltpu.async_copy(w_[b], w_hbm.at[pl.ds(ebase + ci * C, C)], sw_[b])

    def wait_wstore(b):
        pltpu.make_async_copy(w_[b], w_hbm.at[pl.ds(0, C)], sw_[b]).wait()

    def compute(er, rgr, wr, n_edges):
        def grp(g, _):
            sl = pl.ds(g * 16, 16)
            wr[sl] = er[sl] * rgr[sl]
            return 0
        lax.fori_loop(0, n_edges // 16, grp, 0)

    issue_loads(0, 0)
    wait_idx(0)
    issue_gather(0)
    issue_loads(1, 1)

    def body(t2, _):
        a = 2 * t2
        not_last = t2 < _HALF - 1

        wait_e_gather(0)
        wait_idx(1)
        issue_gather(1)

        @pl.when(t2 > 0)
        def _d0():
            wait_wstore(0)
        compute(e0, rg0, w0, C)
        issue_wstore(a, 0)

        @pl.when(not_last)
        def _p0():
            issue_loads(a + 2, 0)

        wait_e_gather(1)

        @pl.when(not_last)
        def _g0():
            wait_idx(0)
            issue_gather(0)

        @pl.when(t2 > 0)
        def _d1():
            wait_wstore(1)
        compute(e1, rg1, w1, C)
        issue_wstore(a + 1, 1)

        @pl.when(not_last)
        def _p1():
            issue_loads(a + 3, 1)
        return 0

    lax.fori_loop(0, _HALF, body, 0)
    wait_wstore(0)
    wait_wstore(1)

    tbase = ebase + FC * C
    pltpu.sync_copy(tgt_hbm.at[pl.ds(tbase, TAIL)], idx_t)
    pltpu.sync_copy(e_hbm.at[pl.ds(tbase, TAIL)], e_t)
    pltpu.sync_copy(rn_sh.at[idx_t], rg_t)
    compute(e_t, rg_t, w_t, TAIL)
    pltpu.sync_copy(w_t, w_hbm.at[pl.ds(tbase, TAIL)])


def _sc_weights(e, tgt, rn_flat):
    f = pl.kernel(
        _sc_w_body,
        out_type=jax.ShapeDtypeStruct((E,), _F32),
        mesh=_mesh(),
        scratch_types=[
            pltpu.VMEM((C,), _I32),
            pltpu.VMEM((C,), _I32),
            pltpu.VMEM((C,), _F32),
            pltpu.VMEM((C,), _F32),
            pltpu.VMEM((C,), _F32),
            pltpu.VMEM((C,), _F32),
            pltpu.VMEM((C,), _F32),
            pltpu.VMEM((C,), _F32),
            pltpu.VMEM((TAIL,), _I32),
            pltpu.VMEM((TAIL,), _F32),
            pltpu.VMEM((TAIL,), _F32),
            pltpu.VMEM((TAIL,), _F32),
        ] + [pltpu.SemaphoreType.DMA] * 8 + [
            pltpu.VMEM_SHARED((PAD_N,), _F32),
        ])
    return f(e, tgt, rn_flat)


def _out_body(a_ref, rn_ref, wv_ref, wo_ref, bo_ref, g_ref, b_ref, o_ref):
    a_n = (a_ref[0] + a_ref[1]) * rn_ref[...]
    t = _dot(a_n, wv_ref[...], ((1,), (0,)))
    proj = _dot(t, wo_ref[...], ((1,), (0,))) + bo_ref[...]
    mu = jnp.mean(proj, axis=1, keepdims=True)
    cen = proj - mu
    var = jnp.mean(cen * cen, axis=1, keepdims=True)
    o_ref[...] = cen * lax.rsqrt(var + 1e-5) * g_ref[...] + b_ref[...]


def _finalize(agg2, rn, Wv, Wo, bo, gamma, beta):
    return pl.pallas_call(
        _out_body,
        grid=(PAD_N // 1024,),
        in_specs=[pl.BlockSpec((NC, 1024, D), lambda i: (0, i, 0)),
                  pl.BlockSpec((1024, 1), lambda i: (i, 0)),
                  pl.BlockSpec((D, D), lambda i: (0, 0)),
                  pl.BlockSpec((D, D), lambda i: (0, 0)),
                  pl.BlockSpec((1, D), lambda i: (0, 0)),
                  pl.BlockSpec((1, D), lambda i: (0, 0)),
                  pl.BlockSpec((1, D), lambda i: (0, 0))],
        out_specs=pl.BlockSpec((1024, D), lambda i: (i, 0)),
        out_shape=jax.ShapeDtypeStruct((PAD_N, D), _F32),
    )(agg2, rn, Wv, Wo, bo.reshape(1, D), gamma.reshape(1, D), beta.reshape(1, D))


def kernel(query_node, key_edge, value_edge, edge_index, Wq, Wk, Wv, Wo, bo, gamma, beta):
    tgt = edge_index[1]
    p = _compute_p(query_node, Wq, Wk)
    e = _sc_edge(p, key_edge, tgt)
    agg2, sums2 = _sc_scatter(e, tgt, value_edge)
    rn = _compute_rn(sums2)
    wts = _sc_weights(e, tgt, rn.reshape(PAD_N))
    out = _finalize(agg2, rn.reshape(PAD_N, 1), Wv, Wo, bo, gamma, beta)
    return out[:N], wts


# speedup vs baseline: 15.9323x; 1.0012x over previous
import jax
import jax.numpy as jnp
from jax import lax
from jax.experimental import pallas as pl
from jax.experimental.pallas import tpu as pltpu
from jax.experimental.pallas import tpu_sc as plsc

N = 10000
E = 320000
D = 128
NC = 2
NS = 16
NW = NC * NS
EP = E // NW
C = 128
FC = EP // C
TAIL = EP - FC * C
PAD_N = 10240
RPS = PAD_N // NS
RPW = PAD_N // NW
SCALE = D ** -0.5

_F32 = jnp.float32
_I32 = jnp.int32


def _mesh():
    return plsc.VectorSubcoreMesh(
        core_axis_name="c", subcore_axis_name="s",
        num_cores=NC, num_subcores=NS)


_DNUMS = lax.GatherDimensionNumbers(
    offset_dims=(), collapsed_slice_dims=(0,), start_index_map=(0,))


def _perm(v, idx):
    return lax.gather(v, idx[:, None], _DNUMS, (1,),
                      mode=lax.GatherScatterMode.PROMISE_IN_BOUNDS)


def _hsum(v, lane):
    for sh in (8, 4, 2, 1):
        v = v + _perm(v, lane ^ sh)
    return v


def _hmax(v, lane):
    for sh in (8, 4, 2, 1):
        v = jnp.maximum(v, _perm(v, lane ^ sh))
    return v


def _dot(a, b, dims):
    return lax.dot_general(
        a, b, (dims, ((), ())),
        precision=lax.Precision.HIGHEST,
        preferred_element_type=_F32)


def _p_body(x_ref, wq_ref, wk_ref, o_ref):
    t = _dot(x_ref[...], wq_ref[...], ((1,), (0,)))
    o_ref[...] = _dot(t, wk_ref[...], ((1,), (1,))) * SCALE


def _compute_p(query, Wq, Wk):
    return pl.pallas_call(
        _p_body,
        grid=(25,),
        in_specs=[pl.BlockSpec((400, D), lambda i: (i, 0)),
                  pl.BlockSpec((D, D), lambda i: (0, 0)),
                  pl.BlockSpec((D, D), lambda i: (0, 0))],
        out_specs=pl.BlockSpec((400, D), lambda i: (i, 0)),
        out_shape=jax.ShapeDtypeStruct((N, D), _F32),
    )(query, Wq, Wk)


_HALF = FC // 2


def _sc_edge_body(p_hbm, key_hbm, tgt_hbm, e_hbm,
                  idx0, idx1, pg0, pg1, key0, key1, e0, e1,
                  idx_t, pg_t, key_t, e_t,
                  si0, si1, sk0, sk1, sg0, sg1, se0, se1):
    cid = lax.axis_index("c")
    sid = lax.axis_index("s")
    w = cid * NS + sid
    ebase = w * EP
    lane = lax.iota(_I32, 16)
    idx_ = (idx0, idx1)
    pg_ = (pg0, pg1)
    key_ = (key0, key1)
    e_ = (e0, e1)
    si_ = (si0, si1)
    sk_ = (sk0, sk1)
    sg_ = (sg0, sg1)
    se_ = (se0, se1)

    def issue_loads(ci, b):
        base = ebase + ci * C
        pltpu.async_copy(tgt_hbm.at[pl.ds(base, C)], idx_[b], si_[b])
        pltpu.async_copy(key_hbm.at[pl.ds(base, C)], key_[b], sk_[b])

    def wait_idx(b):
        pltpu.make_async_copy(tgt_hbm.at[pl.ds(0, C)], idx_[b], si_[b]).wait()

    def issue_gather(b):
        pltpu.async_copy(p_hbm.at[idx_[b]], pg_[b], sg_[b])

    def wait_key_gather(b):
        pltpu.make_async_copy(key_hbm.at[pl.ds(0, C)], key_[b], sk_[b]).wait()
        pltpu.make_async_copy(p_hbm.at[idx_[b]], pg_[b], sg_[b]).wait()

    def issue_estore(ci, b):
        pltpu.async_copy(e_[b], e_hbm.at[pl.ds(ebase + ci * C, C)], se_[b])

    def wait_estore(b):
        pltpu.make_async_copy(e_[b], e_hbm.at[pl.ds(0, C)], se_[b]).wait()

    def compute(pgr, keyr, er, n_edges):
        def sub(sb, svec):
            for c in range(4):
                i = sb * 4 + c
                acc = pgr[i, pl.ds(0, 16)] * keyr[i, pl.ds(0, 16)]
                for j in range(1, 8):
                    acc = acc + pgr[i, pl.ds(16 * j, 16)] * keyr[i, pl.ds(16 * j, 16)]
                svec = jnp.where(lane == (i & 15), _hsum(acc, lane), svec)

            @pl.when((sb & 3) == 3)
            def _store():
                er[pl.ds((sb >> 2) * 16, 16)] = jnp.exp(jnp.minimum(svec, 60.0))
            return jnp.where((sb & 3) == 3, jnp.zeros((16,), _F32), svec)

        lax.fori_loop(0, n_edges // 4, sub, jnp.zeros((16,), _F32))

    issue_loads(0, 0)
    wait_idx(0)
    issue_gather(0)
    issue_loads(1, 1)

    def body(t2, _):
        a = 2 * t2
        not_last = t2 < _HALF - 1

        wait_key_gather(0)
        wait_idx(1)
        issue_gather(1)

        @pl.when(t2 > 0)
        def _d0():
            wait_estore(0)
        compute(pg0, key0, e0, C)
        issue_estore(a, 0)

        @pl.when(not_last)
        def _p0():
            issue_loads(a + 2, 0)

        wait_key_gather(1)

        @pl.when(not_last)
        def _g0():
            wait_idx(0)
            issue_gather(0)

        @pl.when(t2 > 0)
        def _d1():
            wait_estore(1)
        compute(pg1, key1, e1, C)
        issue_estore(a + 1, 1)

        @pl.when(not_last)
        def _p1():
            issue_loads(a + 3, 1)
        return 0

    lax.fori_loop(0, _HALF, body, 0)
    wait_estore(0)
    wait_estore(1)

    tbase = ebase + FC * C
    pltpu.sync_copy(tgt_hbm.at[pl.ds(tbase, TAIL)], idx_t)
    pltpu.sync_copy(key_hbm.at[pl.ds(tbase, TAIL)], key_t)
    pltpu.sync_copy(p_hbm.at[idx_t], pg_t)
    compute(pg_t, key_t, e_t, TAIL)
    pltpu.sync_copy(e_t, e_hbm.at[pl.ds(tbase, TAIL)])


def _sc_edge(p, key_edge, tgt):
    f = pl.kernel(
        _sc_edge_body,
        out_type=jax.ShapeDtypeStruct((E,), _F32),
        mesh=_mesh(),
        scratch_types=[
            pltpu.VMEM((C,), _I32),
            pltpu.VMEM((C,), _I32),
            pltpu.VMEM((C, D), _F32),
            pltpu.VMEM((C, D), _F32),
            pltpu.VMEM((C, D), _F32),
            pltpu.VMEM((C, D), _F32),
            pltpu.VMEM((C,), _F32),
            pltpu.VMEM((C,), _F32),
            pltpu.VMEM((TAIL,), _I32),
            pltpu.VMEM((TAIL, D), _F32),
            pltpu.VMEM((TAIL, D), _F32),
            pltpu.VMEM((TAIL,), _F32),
        ] + [pltpu.SemaphoreType.DMA] * 8)
    return f(p, key_edge, tgt)


def _sc_scatter_body(e_hbm, tgt_hbm, val_hbm,
                     agg_hbm, sums_hbm,
                     idx0, idx1, e0, e1, val0, val1,
                     idx_t, e_t, val_t,
                     zrow_v, zsum_v,
                     si0, si1, se0, se1, sv0, sv1, sa0, sa1, ss0, ss1,
                     agg_sh, sums_sh):
    cid = lax.axis_index("c")
    sid = lax.axis_index("s")
    w = cid * NS + sid
    ebase = w * EP
    lane = lax.iota(_I32, 16)
    idx_ = (idx0, idx1)
    e_ = (e0, e1)
    val_ = (val0, val1)
    si_ = (si0, si1)
    se_ = (se0, se1)
    sv_ = (sv0, sv1)
    sa_ = (sa0, sa1)
    ss_ = (ss0, ss1)

    def zr(r, _):
        for j in range(8):
            zrow_v[r, pl.ds(16 * j, 16)] = jnp.zeros((16,), _F32)
        return 0
    lax.fori_loop(0, 40, zr, 0)

    def zs(i, _):
        zsum_v[pl.ds(16 * i, 16)] = jnp.zeros((16,), _F32)
        return 0
    lax.fori_loop(0, RPS // 16, zs, 0)

    for k2 in range(RPS // 40):
        pltpu.sync_copy(zrow_v, agg_sh.at[pl.ds(sid * RPS + k2 * 40, 40)])
    pltpu.sync_copy(zsum_v, sums_sh.at[pl.ds(sid * RPS, RPS)])
    plsc.subcore_barrier()

    def issue_loads(ci, b):
        base = ebase + ci * C
        pltpu.async_copy(tgt_hbm.at[pl.ds(base, C)], idx_[b], si_[b])
        pltpu.async_copy(e_hbm.at[pl.ds(base, C)], e_[b], se_[b])
        pltpu.async_copy(val_hbm.at[pl.ds(base, C)], val_[b], sv_[b])

    def wait_loads(b):
        pltpu.make_async_copy(tgt_hbm.at[pl.ds(0, C)], idx_[b], si_[b]).wait()
        pltpu.make_async_copy(e_hbm.at[pl.ds(0, C)], e_[b], se_[b]).wait()
        pltpu.make_async_copy(val_hbm.at[pl.ds(0, C)], val_[b], sv_[b]).wait()

    def scale(er, vr, n_edges):
        def grp(g, _):
            ev16 = er[pl.ds(g * 16, 16)]
            for u in range(16):
                i = g * 16 + u
                wb = _perm(ev16, jnp.full((16,), u, _I32))
                for j in range(8):
                    slj = pl.ds(16 * j, 16)
                    vr[i, slj] = vr[i, slj] * wb
            return 0
        lax.fori_loop(0, n_edges // 16, grp, 0)

    def issue_scatters(b):
        pltpu.async_copy(val_[b], agg_sh.at[idx_[b]], sa_[b], add=True)
        pltpu.async_copy(e_[b], sums_sh.at[idx_[b]], ss_[b], add=True)

    def wait_scatters(b):
        pltpu.make_async_copy(val_[b], agg_sh.at[idx_[b]], sa_[b]).wait()
        pltpu.make_async_copy(e_[b], sums_sh.at[idx_[b]], ss_[b]).wait()

    issue_loads(0, 0)
    issue_loads(1, 1)

    def body(t2, _):
        a = 2 * t2
        not_last = t2 < _HALF - 1

        wait_loads(0)
        scale(e0, val0, C)
        issue_scatters(0)

        wait_loads(1)
        scale(e1, val1, C)
        issue_scatters(1)

        @pl.when(not_last)
        def _p():
            wait_scatters(0)
            issue_loads(a + 2, 0)
            wait_scatters(1)
            issue_loads(a + 3, 1)
        return 0

    lax.fori_loop(0, _HALF, body, 0)
    wait_scatters(0)
    wait_scatters(1)

    tbase = ebase + FC * C
    pltpu.sync_copy(tgt_hbm.at[pl.ds(tbase, TAIL)], idx_t)
    pltpu.sync_copy(e_hbm.at[pl.ds(tbase, TAIL)], e_t)
    pltpu.sync_copy(val_hbm.at[pl.ds(tbase, TAIL)], val_t)
    scale(e_t, val_t, TAIL)
    pltpu.sync_copy(val_t, agg_sh.at[idx_t], add=True)
    pltpu.sync_copy(e_t, sums_sh.at[idx_t], add=True)

    plsc.subcore_barrier()
    pltpu.sync_copy(agg_sh.at[pl.ds(sid * RPS, RPS)],
                    agg_hbm.at[cid, pl.ds(sid * RPS, RPS)])
    pltpu.sync_copy(sums_sh.at[pl.ds(sid * RPS, RPS)],
                    sums_hbm.at[cid, pl.ds(sid * RPS, RPS)])


def _sc_scatter(e, tgt, value_edge):
    f = pl.kernel(
        _sc_scatter_body,
        out_type=(jax.ShapeDtypeStruct((NC, PAD_N, D), _F32),
                  jax.ShapeDtypeStruct((NC, PAD_N), _F32)),
        mesh=_mesh(),
        scratch_types=[
            pltpu.VMEM((C,), _I32),
            pltpu.VMEM((C,), _I32),
            pltpu.VMEM((C,), _F32),
            pltpu.VMEM((C,), _F32),
            pltpu.VMEM((C, D), _F32),
            pltpu.VMEM((C, D), _F32),
            pltpu.VMEM((TAIL,), _I32),
            pltpu.VMEM((TAIL,), _F32),
            pltpu.VMEM((TAIL, D), _F32),
            pltpu.VMEM((40, D), _F32),
            pltpu.VMEM((RPS,), _F32),
        ] + [pltpu.SemaphoreType.DMA] * 10 + [
            pltpu.VMEM_SHARED((PAD_N, D), _F32),
            pltpu.VMEM_SHARED((PAD_N,), _F32),
        ])
    return f(e, tgt, value_edge)


def _rn_body(s_ref, o_ref):
    o_ref[...] = 1.0 / (s_ref[0] + s_ref[1] + 1e-12)


def _compute_rn(sums2):
    return pl.pallas_call(
        _rn_body,
        out_shape=jax.ShapeDtypeStruct((PAD_N // D, D), _F32),
    )(sums2.reshape(NC, PAD_N // D, D))


def _sc_w_body(e_hbm, tgt_hbm, sums_hbm, w_hbm,
               idx0, idx1, e0, e1, rg0, rg1, w0, w1,
               idx_t, e_t, rg_t, w_t, s0_v, s1_v,
               si0, si1, se0, se1, sg0, sg1, sw0, sw1,
               rn_sh):
    cid = lax.axis_index("c")
    sid = lax.axis_index("s")
    w = cid * NS + sid
    ebase = w * EP
    idx_ = (idx0, idx1)
    e_ = (e0, e1)
    rg_ = (rg0, rg1)
    w_ = (w0, w1)
    si_ = (si0, si1)
    se_ = (se0, se1)
    sg_ = (sg0, sg1)
    sw_ = (sw0, sw1)

    pltpu.sync_copy(sums_hbm.at[0, pl.ds(sid * RPS, RPS)], s0_v)
    pltpu.sync_copy(sums_hbm.at[1, pl.ds(sid * RPS, RPS)], s1_v)

    def rncomb(i, _):
        sl = pl.ds(16 * i, 16)
        s0_v[sl] = 1.0 / (s0_v[sl] + s1_v[sl] + 1e-12)
        return 0
    lax.fori_loop(0, RPS // 16, rncomb, 0)
    pltpu.sync_copy(s0_v, rn_sh.at[pl.ds(sid * RPS, RPS)])
    plsc.subcore_barrier()

    def issue_loads(ci, b):
        base = ebase + ci * C
        pltpu.async_copy(tgt_hbm.at[pl.ds(base, C)], idx_[b], si_[b])
        pltpu.async_copy(e_hbm.at[pl.ds(base, C)], e_[b], se_[b])

    def wait_idx(b):
        pltpu.make_async_copy(tgt_hbm.at[pl.ds(0, C)], idx_[b], si_[b]).wait()

    def issue_gather(b):
        pltpu.async_copy(rn_sh.at[idx_[b]], rg_[b], sg_[b])

    def wait_e_gather(b):
        pltpu.make_async_copy(e_hbm.at[pl.ds(0, C)], e_[b], se_[b]).wait()
        pltpu.make_async_copy(rn_sh.at[idx_[b]], rg_[b], sg_[b]).wait()

    def issue_wstore(ci, b):
        pltpu.async_copy(w_[b], w_hbm.at[pl.ds(ebase + ci * C, C)], sw_[b])

    def wait_wstore(b):
        pltpu.make_async_copy(w_[b], w_hbm.at[pl.ds(0, C)], sw_[b]).wait()

    def compute(er, rgr, wr, n_edges):
        def grp(g, _):
            sl = pl.ds(g * 16, 16)
            wr[sl] = er[sl] * rgr[sl]
            return 0
        lax.fori_loop(0, n_edges // 16, grp, 0)

    issue_loads(0, 0)
    wait_idx(0)
    issue_gather(0)
    issue_loads(1, 1)

    def body(t2, _):
        a = 2 * t2
        not_last = t2 < _HALF - 1

        wait_e_gather(0)
        wait_idx(1)
        issue_gather(1)

        @pl.when(t2 > 0)
        def _d0():
            wait_wstore(0)
        compute(e0, rg0, w0, C)
        issue_wstore(a, 0)

        @pl.when(not_last)
        def _p0():
            issue_loads(a + 2, 0)

        wait_e_gather(1)

        @pl.when(not_last)
        def _g0():
            wait_idx(0)
            issue_gather(0)

        @pl.when(t2 > 0)
        def _d1():
            wait_wstore(1)
        compute(e1, rg1, w1, C)
        issue_wstore(a + 1, 1)

        @pl.when(not_last)
        def _p1():
            issue_loads(a + 3, 1)
        return 0

    lax.fori_loop(0, _HALF, body, 0)
    wait_wstore(0)
    wait_wstore(1)

    tbase = ebase + FC * C
    pltpu.sync_copy(tgt_hbm.at[pl.ds(tbase, TAIL)], idx_t)
    pltpu.sync_copy(e_hbm.at[pl.ds(tbase, TAIL)], e_t)
    pltpu.sync_copy(rn_sh.at[idx_t], rg_t)
    compute(e_t, rg_t, w_t, TAIL)
    pltpu.sync_copy(w_t, w_hbm.at[pl.ds(tbase, TAIL)])


def _sc_weights(e, tgt, sums2):
    f = pl.kernel(
        _sc_w_body,
        out_type=jax.ShapeDtypeStruct((E,), _F32),
        mesh=_mesh(),
        scratch_types=[
            pltpu.VMEM((C,), _I32),
            pltpu.VMEM((C,), _I32),
            pltpu.VMEM((C,), _F32),
            pltpu.VMEM((C,), _F32),
            pltpu.VMEM((C,), _F32),
            pltpu.VMEM((C,), _F32),
            pltpu.VMEM((C,), _F32),
            pltpu.VMEM((C,), _F32),
            pltpu.VMEM((TAIL,), _I32),
            pltpu.VMEM((TAIL,), _F32),
            pltpu.VMEM((TAIL,), _F32),
            pltpu.VMEM((TAIL,), _F32),
            pltpu.VMEM((RPS,), _F32),
            pltpu.VMEM((RPS,), _F32),
        ] + [pltpu.SemaphoreType.DMA] * 8 + [
            pltpu.VMEM_SHARED((PAD_N,), _F32),
        ])
    return f(e, tgt, sums2)


def _out_body(a_ref, rn_ref, wv_ref, wo_ref, bo_ref, g_ref, b_ref, o_ref):
    a_n = (a_ref[0] + a_ref[1]) * rn_ref[...]
    t = _dot(a_n, wv_ref[...], ((1,), (0,)))
    proj = _dot(t, wo_ref[...], ((1,), (0,))) + bo_ref[...]
    mu = jnp.mean(proj, axis=1, keepdims=True)
    cen = proj - mu
    var = jnp.mean(cen * cen, axis=1, keepdims=True)
    o_ref[...] = cen * lax.rsqrt(var + 1e-5) * g_ref[...] + b_ref[...]


def _finalize(agg2, rn, Wv, Wo, bo, gamma, beta):
    return pl.pallas_call(
        _out_body,
        grid=(PAD_N // 1024,),
        in_specs=[pl.BlockSpec((NC, 1024, D), lambda i: (0, i, 0)),
                  pl.BlockSpec((1024, 1), lambda i: (i, 0)),
                  pl.BlockSpec((D, D), lambda i: (0, 0)),
                  pl.BlockSpec((D, D), lambda i: (0, 0)),
                  pl.BlockSpec((1, D), lambda i: (0, 0)),
                  pl.BlockSpec((1, D), lambda i: (0, 0)),
                  pl.BlockSpec((1, D), lambda i: (0, 0))],
        out_specs=pl.BlockSpec((1024, D), lambda i: (i, 0)),
        out_shape=jax.ShapeDtypeStruct((PAD_N, D), _F32),
    )(agg2, rn, Wv, Wo, bo.reshape(1, D), gamma.reshape(1, D), beta.reshape(1, D))


def kernel(query_node, key_edge, value_edge, edge_index, Wq, Wk, Wv, Wo, bo, gamma, beta):
    tgt = edge_index[1]
    p = _compute_p(query_node, Wq, Wk)
    e = _sc_edge(p, key_edge, tgt)
    agg2, sums2 = _sc_scatter(e, tgt, value_edge)
    rn = _compute_rn(sums2)
    wts = _sc_weights(e, tgt, sums2)
    out = _finalize(agg2, rn.reshape(PAD_N, 1), Wv, Wo, bo, gamma, beta)
    return out[:N], wts
